# Initial kernel scaffold; baseline (speedup 1.0000x reference)
import jax
import jax.numpy as jnp
from jax.experimental import pallas as pl


def kernel(x, edge_attr, u, edge_index, batch, We1, be1, We2, be2, Wn1, bn1, Wn2, bn2, Wg1, bg1, Wg2, bg2):
    raise NotImplementedError("write your pallas kernel here")


# speedup vs baseline: 3.5322x; 3.5322x over previous
import functools

import jax
import jax.numpy as jnp
from jax import lax
from jax.experimental import pallas as pl
from jax.experimental.pallas import tpu as pltpu
from jax.experimental.pallas import tpu_sc as plsc

N = 10000
E = 160000
D = 256

NC = 2
NS = 16
NW = NC * NS

CH = 128
NCHUNK = E // CH
KG = -(-NCHUNK // NW)
KS = -(-NCHUNK // NS)
DH = D // NC
ROWB = 640
NPAD = NS * ROWB

_MESH = dict(core_axis_name="c", subcore_axis_name="s", num_cores=NC,
             num_subcores=NS)


def _gather_body(pa_hbm, pb_hbm, src_hbm, dst_hbm, g1_hbm, g2_hbm,
                 idxs_v, idxd_v, rows1_v, rows2_v, sem1, sem2):
    wid = lax.axis_index("s") * NC + lax.axis_index("c")

    def step(k, carry):
        c = wid + k * NW

        @pl.when(c < NCHUNK)
        def _():
            off = pl.multiple_of(c * CH, CH)
            pltpu.sync_copy(src_hbm.at[c], idxs_v)
            pltpu.sync_copy(dst_hbm.at[c], idxd_v)
            cp1 = pltpu.async_copy(pa_hbm.at[idxs_v], rows1_v, sem1)
            cp2 = pltpu.async_copy(pb_hbm.at[idxd_v], rows2_v, sem2)
            cp1.wait()
            pltpu.sync_copy(rows1_v, g1_hbm.at[pl.ds(off, CH)])
            cp2.wait()
            pltpu.sync_copy(rows2_v, g2_hbm.at[pl.ds(off, CH)])

        return carry

    lax.fori_loop(0, KG, step, 0)


_gather_call = functools.partial(
    pl.kernel,
    out_type=[jax.ShapeDtypeStruct((E, D), jnp.float32),
              jax.ShapeDtypeStruct((E, D), jnp.float32)],
    mesh=plsc.VectorSubcoreMesh(**_MESH),
    scratch_types=[
        pltpu.VMEM((CH,), jnp.int32),
        pltpu.VMEM((CH,), jnp.int32),
        pltpu.VMEM((CH, D), jnp.float32),
        pltpu.VMEM((CH, D), jnp.float32),
        pltpu.SemaphoreType.DMA,
        pltpu.SemaphoreType.DMA,
    ],
)(_gather_body)


def _scatter_body(enew_hbm, dst_hbm, zeros_hbm, agg_hbm, acc_sh, idx_v, pay_v):
    cid = lax.axis_index("c")
    sid = lax.axis_index("s")
    coff = pl.multiple_of(cid * DH, DH)
    roff = pl.multiple_of(sid * ROWB, ROWB)
    pltpu.sync_copy(zeros_hbm, acc_sh.at[pl.ds(roff, ROWB)])
    plsc.subcore_barrier()

    def sstep(k, carry):
        c = sid + k * NS

        @pl.when(c < NCHUNK)
        def _():
            pltpu.sync_copy(dst_hbm.at[c], idx_v)
            pltpu.sync_copy(
                enew_hbm.at[pl.ds(pl.multiple_of(c * CH, CH), CH),
                            pl.ds(coff, DH)],
                pay_v)
            pltpu.sync_copy(pay_v, acc_sh.at[idx_v], add=True)

        return carry

    lax.fori_loop(0, KS, sstep, 0)
    plsc.subcore_barrier()

    @pl.when(sid < NS - 1)
    def _():
        pltpu.sync_copy(acc_sh.at[pl.ds(roff, ROWB)],
                        agg_hbm.at[pl.ds(roff, ROWB), pl.ds(coff, DH)])

    @pl.when(sid == NS - 1)
    def _():
        pltpu.sync_copy(acc_sh.at[pl.ds((NS - 1) * ROWB, N - (NS - 1) * ROWB)],
                        agg_hbm.at[pl.ds((NS - 1) * ROWB, N - (NS - 1) * ROWB),
                                   pl.ds(coff, DH)])


_scatter_call = functools.partial(
    pl.kernel,
    out_type=jax.ShapeDtypeStruct((N, D), jnp.float32),
    mesh=plsc.VectorSubcoreMesh(**_MESH),
    scratch_types=[
        pltpu.VMEM_SHARED((NPAD, DH), jnp.float32),
        pltpu.VMEM((CH,), jnp.int32),
        pltpu.VMEM((CH, DH), jnp.float32),
    ],
)(_scatter_body)


BN = 1000
GN = N // BN


def _proj_body(x_ref, w_ref, pa_ref, pb_ref, qa_ref):
    p = jnp.dot(x_ref[...], w_ref[...], preferred_element_type=jnp.float32)
    pa_ref[...] = p[:, :D]
    pb_ref[...] = p[:, D:2 * D]
    qa_ref[...] = p[:, 2 * D:]


def _proj_call(x, wcat):
    blk = lambda i: (i, 0)
    out = jax.ShapeDtypeStruct((N, D), jnp.float32)
    return pl.pallas_call(
        _proj_body,
        grid=(GN,),
        in_specs=[
            pl.BlockSpec((BN, D), blk),
            pl.BlockSpec((D, 3 * D), lambda i: (0, 0)),
        ],
        out_specs=[pl.BlockSpec((BN, D), blk)] * 3,
        out_shape=[out, out, out],
    )(x, wcat)


BE = 1000
GE = E // BE


def _edge_body(u8_ref, we1c_ref, we1d_ref, be1_ref, we2_ref, be2_ref,
               g1_ref, g2_ref, ea_ref, enew_ref, esum_ref):
    i = pl.program_id(0)
    ea = ea_ref[...]
    cst = jnp.dot(u8_ref[...], we1d_ref[...], preferred_element_type=jnp.float32)
    h = g1_ref[...] + g2_ref[...] + jnp.dot(ea, we1c_ref[...],
                                            preferred_element_type=jnp.float32)
    h = jnp.maximum(h + cst[0:1, :] + be1_ref[...], 0.0)
    enew = ea + jnp.dot(h, we2_ref[...],
                        preferred_element_type=jnp.float32) + be2_ref[...]
    enew_ref[...] = enew

    @pl.when(i == 0)
    def _():
        esum_ref[...] = jnp.zeros_like(esum_ref)

    esum_ref[...] += jnp.sum(enew, axis=0, keepdims=True)


def _edge_call(u8, we1c, we1d, be1, we2, be2, g1, g2, ea):
    blk = lambda i: (i, 0)
    fixed = lambda i: (0, 0)
    return pl.pallas_call(
        _edge_body,
        grid=(GE,),
        in_specs=[
            pl.BlockSpec((8, D), fixed),
            pl.BlockSpec((D, D), fixed),
            pl.BlockSpec((D, D), fixed),
            pl.BlockSpec((1, D), fixed),
            pl.BlockSpec((D, D), fixed),
            pl.BlockSpec((1, D), fixed),
            pl.BlockSpec((BE, D), blk),
            pl.BlockSpec((BE, D), blk),
            pl.BlockSpec((BE, D), blk),
        ],
        out_specs=[pl.BlockSpec((BE, D), blk), pl.BlockSpec((1, D), fixed)],
        out_shape=[jax.ShapeDtypeStruct((E, D), jnp.float32),
                   jax.ShapeDtypeStruct((1, D), jnp.float32)],
    )(u8, we1c, we1d, be1, we2, be2, g1, g2, ea)


def _node_body(u8_ref, wn1b_ref, wn1c_ref, bn1_ref, wn2_ref, bn2_ref,
               esum_ref, wg1_ref, bg1_ref, wg2_ref, bg2_ref,
               x_ref, agg_ref, qa_ref, xnew_ref, unew_ref, nsum_acc):
    i = pl.program_id(0)
    cst = jnp.dot(u8_ref[...], wn1c_ref[...], preferred_element_type=jnp.float32)
    h = qa_ref[...] + jnp.dot(agg_ref[...], wn1b_ref[...],
                              preferred_element_type=jnp.float32)
    h = jnp.maximum(h + cst[0:1, :] + bn1_ref[...], 0.0)
    xn = x_ref[...] + jnp.dot(h, wn2_ref[...],
                              preferred_element_type=jnp.float32) + bn2_ref[...]
    xnew_ref[...] = xn

    @pl.when(i == 0)
    def _():
        nsum_acc[...] = jnp.zeros_like(nsum_acc)

    nsum_acc[...] += jnp.sum(xn, axis=0, keepdims=True)

    @pl.when(i == GN - 1)
    def _():
        nmean = jnp.broadcast_to(nsum_acc[...] * (1.0 / N), (8, D))
        emean = jnp.broadcast_to(esum_ref[...] * (1.0 / E), (8, D))
        g8 = jnp.concatenate([nmean, emean, u8_ref[...]], axis=1)
        hg = jnp.maximum(
            jnp.dot(g8, wg1_ref[...], preferred_element_type=jnp.float32)
            + bg1_ref[...], 0.0)
        un = jnp.dot(hg, wg2_ref[...],
                     preferred_element_type=jnp.float32) + bg2_ref[...]
        unew_ref[...] = u8_ref[0:1, :] + un[0:1, :]


def _node_call(u8, wn1b, wn1c, bn1, wn2, bn2, esum, wg1, bg1, wg2, bg2,
               x, agg, qa):
    blk = lambda i: (i, 0)
    fixed = lambda i: (0, 0)
    return pl.pallas_call(
        _node_body,
        grid=(GN,),
        in_specs=[
            pl.BlockSpec((8, D), fixed),
            pl.BlockSpec((D, D), fixed),
            pl.BlockSpec((D, D), fixed),
            pl.BlockSpec((1, D), fixed),
            pl.BlockSpec((D, D), fixed),
            pl.BlockSpec((1, D), fixed),
            pl.BlockSpec((1, D), fixed),
            pl.BlockSpec((3 * D, D), fixed),
            pl.BlockSpec((1, D), fixed),
            pl.BlockSpec((D, D), fixed),
            pl.BlockSpec((1, D), fixed),
            pl.BlockSpec((BN, D), blk),
            pl.BlockSpec((BN, D), blk),
            pl.BlockSpec((BN, D), blk),
        ],
        out_specs=[pl.BlockSpec((BN, D), blk), pl.BlockSpec((1, D), fixed)],
        out_shape=[jax.ShapeDtypeStruct((N, D), jnp.float32),
                   jax.ShapeDtypeStruct((1, D), jnp.float32)],
        scratch_shapes=[pltpu.VMEM((1, D), jnp.float32)],
    )(u8, wn1b, wn1c, bn1, wn2, bn2, esum, wg1, bg1, wg2, bg2, x, agg, qa)


def kernel(x, edge_attr, u, edge_index, batch,
           We1, be1, We2, be2,
           Wn1, bn1, Wn2, bn2,
           Wg1, bg1, Wg2, bg2):
    src = edge_index[0].astype(jnp.int32)
    dst = edge_index[1].astype(jnp.int32)
    src2 = src.reshape(NCHUNK, CH)
    dst2 = dst.reshape(NCHUNK, CH)

    wcat = jnp.concatenate([We1[:D], We1[D:2 * D], Wn1[:D]], axis=1)
    u8 = jnp.broadcast_to(u, (8, D))
    be1r = be1.reshape(1, D)
    be2r = be2.reshape(1, D)
    bn1r = bn1.reshape(1, D)
    bn2r = bn2.reshape(1, D)
    bg1r = bg1.reshape(1, D)
    bg2r = bg2.reshape(1, D)
    zeros = jnp.zeros((ROWB, DH), jnp.float32)

    pa, pb, qa = _proj_call(x, wcat)
    g1, g2 = _gather_call(pa, pb, src2, dst2)
    edge_new, esum = _edge_call(u8, We1[2 * D:3 * D], We1[3 * D:], be1r,
                                We2, be2r, g1, g2, edge_attr)
    agg = _scatter_call(edge_new, dst2, zeros)
    x_new, u_new = _node_call(u8, Wn1[D:2 * D], Wn1[2 * D:], bn1r, Wn2, bn2r,
                              esum, Wg1, bg1r, Wg2, bg2r, x, agg, qa)
    return x_new, edge_new, u_new


# speedup vs baseline: 4.2403x; 1.2005x over previous
import functools

import jax
import jax.numpy as jnp
from jax import lax
from jax.experimental import pallas as pl
from jax.experimental.pallas import tpu as pltpu
from jax.experimental.pallas import tpu_sc as plsc

N = 10000
E = 160000
D = 256

NC = 2
NS = 16
NW = NC * NS

CH = 128
NCHUNK = E // CH
KG = -(-NCHUNK // NW)
KS = -(-NCHUNK // NS)
DH = D // NC
ROWB = 640
NPAD = NS * ROWB

_MESH = dict(core_axis_name="c", subcore_axis_name="s", num_cores=NC,
             num_subcores=NS)


def _gather_body(pa_hbm, pb_hbm, src_hbm, dst_hbm, g1_hbm, g2_hbm,
                 idxs_v, idxd_v, rows1_v, rows2_v, sem1, sem2):
    wid = lax.axis_index("s") * NC + lax.axis_index("c")

    def step(k, carry):
        c = wid + k * NW

        @pl.when(c < NCHUNK)
        def _():
            off = pl.multiple_of(c * CH, CH)
            pltpu.sync_copy(src_hbm.at[c], idxs_v)
            pltpu.sync_copy(dst_hbm.at[c], idxd_v)
            cp1 = pltpu.async_copy(pa_hbm.at[idxs_v], rows1_v, sem1)
            cp2 = pltpu.async_copy(pb_hbm.at[idxd_v], rows2_v, sem2)
            cp1.wait()
            pltpu.sync_copy(rows1_v, g1_hbm.at[pl.ds(off, CH)])
            cp2.wait()
            pltpu.sync_copy(rows2_v, g2_hbm.at[pl.ds(off, CH)])

        return carry

    lax.fori_loop(0, KG, step, 0)


_gather_call = functools.partial(
    pl.kernel,
    out_type=[jax.ShapeDtypeStruct((E, D // 2), jnp.int32),
              jax.ShapeDtypeStruct((E, D // 2), jnp.int32)],
    mesh=plsc.VectorSubcoreMesh(**_MESH),
    scratch_types=[
        pltpu.VMEM((CH,), jnp.int32),
        pltpu.VMEM((CH,), jnp.int32),
        pltpu.VMEM((CH, D // 2), jnp.int32),
        pltpu.VMEM((CH, D // 2), jnp.int32),
        pltpu.SemaphoreType.DMA,
        pltpu.SemaphoreType.DMA,
    ],
)(_gather_body)


def _scatter_body(enew_hbm, dst_hbm, zeros_hbm, agg_hbm, acc_sh, idx_v, pay_v):
    cid = lax.axis_index("c")
    sid = lax.axis_index("s")
    coff = pl.multiple_of(cid * DH, DH)
    roff = pl.multiple_of(sid * ROWB, ROWB)
    pltpu.sync_copy(zeros_hbm, acc_sh.at[pl.ds(roff, ROWB)])
    plsc.subcore_barrier()

    def sstep(k, carry):
        c = sid + k * NS

        @pl.when(c < NCHUNK)
        def _():
            pltpu.sync_copy(dst_hbm.at[c], idx_v)
            pltpu.sync_copy(
                enew_hbm.at[pl.ds(pl.multiple_of(c * CH, CH), CH),
                            pl.ds(coff, DH)],
                pay_v)
            pltpu.sync_copy(pay_v, acc_sh.at[idx_v], add=True)

        return carry

    lax.fori_loop(0, KS, sstep, 0)
    plsc.subcore_barrier()

    @pl.when(sid < NS - 1)
    def _():
        pltpu.sync_copy(acc_sh.at[pl.ds(roff, ROWB)],
                        agg_hbm.at[pl.ds(roff, ROWB), pl.ds(coff, DH)])

    @pl.when(sid == NS - 1)
    def _():
        pltpu.sync_copy(acc_sh.at[pl.ds((NS - 1) * ROWB, N - (NS - 1) * ROWB)],
                        agg_hbm.at[pl.ds((NS - 1) * ROWB, N - (NS - 1) * ROWB),
                                   pl.ds(coff, DH)])


_scatter_call = functools.partial(
    pl.kernel,
    out_type=jax.ShapeDtypeStruct((N, D), jnp.float32),
    mesh=plsc.VectorSubcoreMesh(**_MESH),
    scratch_types=[
        pltpu.VMEM_SHARED((NPAD, DH), jnp.float32),
        pltpu.VMEM((CH,), jnp.int32),
        pltpu.VMEM((CH, DH), jnp.float32),
    ],
)(_scatter_body)


BN = 1000
GN = N // BN


def _pack_bf16_pair(lo_f32, hi_f32):
    tl = lax.bitcast_convert_type(lo_f32, jnp.uint32) + jnp.uint32(0x8000)
    th = lax.bitcast_convert_type(hi_f32, jnp.uint32) + jnp.uint32(0x8000)
    packed = lax.shift_right_logical(tl, jnp.uint32(16)) | (th & jnp.uint32(0xFFFF0000))
    return lax.bitcast_convert_type(packed, jnp.int32)


def _unpack_bf16_pair(packed_i32):
    u = lax.bitcast_convert_type(packed_i32, jnp.uint32)
    lo = lax.bitcast_convert_type(lax.shift_left(u, jnp.uint32(16)), jnp.float32)
    hi = lax.bitcast_convert_type(u & jnp.uint32(0xFFFF0000), jnp.float32)
    return lo, hi


def _proj_body(x_ref, w_ref, pa_ref, pb_ref, qa_ref):
    p = jnp.dot(x_ref[...], w_ref[...], preferred_element_type=jnp.float32)
    pa_ref[...] = _pack_bf16_pair(p[:, :D // 2], p[:, D // 2:D])
    pb_ref[...] = _pack_bf16_pair(p[:, D:D + D // 2], p[:, D + D // 2:2 * D])
    qa_ref[...] = p[:, 2 * D:]


def _proj_call(x, wcat):
    blk = lambda i: (i, 0)
    outp = jax.ShapeDtypeStruct((N, D // 2), jnp.int32)
    return pl.pallas_call(
        _proj_body,
        grid=(GN,),
        in_specs=[
            pl.BlockSpec((BN, D), blk),
            pl.BlockSpec((D, 3 * D), lambda i: (0, 0)),
        ],
        out_specs=[pl.BlockSpec((BN, D // 2), blk),
                   pl.BlockSpec((BN, D // 2), blk),
                   pl.BlockSpec((BN, D), blk)],
        out_shape=[outp, outp, jax.ShapeDtypeStruct((N, D), jnp.float32)],
    )(x, wcat)


BE = 1000
GE = E // BE


def _edge_body(u8_ref, we1c_ref, we1d_ref, be1_ref, we2_ref, be2_ref,
               g1_ref, g2_ref, ea_ref, enew_ref, esum_ref):
    i = pl.program_id(0)
    ea = ea_ref[...]
    cst = jnp.dot(u8_ref[...], we1d_ref[...], preferred_element_type=jnp.float32)
    lo1, hi1 = _unpack_bf16_pair(g1_ref[...])
    lo2, hi2 = _unpack_bf16_pair(g2_ref[...])
    g = jnp.concatenate([lo1 + lo2, hi1 + hi2], axis=-1)
    h = g + jnp.dot(ea, we1c_ref[...], preferred_element_type=jnp.float32)
    h = jnp.maximum(h + cst[0:1, :] + be1_ref[...], 0.0)
    enew = ea + jnp.dot(h, we2_ref[...],
                        preferred_element_type=jnp.float32) + be2_ref[...]
    enew_ref[...] = enew

    @pl.when(i == 0)
    def _():
        esum_ref[...] = jnp.zeros_like(esum_ref)

    esum_ref[...] += jnp.sum(enew, axis=0, keepdims=True)


def _edge_call(u8, we1c, we1d, be1, we2, be2, g1, g2, ea):
    blk = lambda i: (i, 0)
    fixed = lambda i: (0, 0)
    return pl.pallas_call(
        _edge_body,
        grid=(GE,),
        in_specs=[
            pl.BlockSpec((8, D), fixed),
            pl.BlockSpec((D, D), fixed),
            pl.BlockSpec((D, D), fixed),
            pl.BlockSpec((1, D), fixed),
            pl.BlockSpec((D, D), fixed),
            pl.BlockSpec((1, D), fixed),
            pl.BlockSpec((BE, D // 2), blk),
            pl.BlockSpec((BE, D // 2), blk),
            pl.BlockSpec((BE, D), blk),
        ],
        out_specs=[pl.BlockSpec((BE, D), blk), pl.BlockSpec((1, D), fixed)],
        out_shape=[jax.ShapeDtypeStruct((E, D), jnp.float32),
                   jax.ShapeDtypeStruct((1, D), jnp.float32)],
    )(u8, we1c, we1d, be1, we2, be2, g1, g2, ea)


def _node_body(u8_ref, wn1b_ref, wn1c_ref, bn1_ref, wn2_ref, bn2_ref,
               esum_ref, wg1_ref, bg1_ref, wg2_ref, bg2_ref,
               x_ref, agg_ref, qa_ref, xnew_ref, unew_ref, nsum_acc):
    i = pl.program_id(0)
    cst = jnp.dot(u8_ref[...], wn1c_ref[...], preferred_element_type=jnp.float32)
    h = qa_ref[...] + jnp.dot(agg_ref[...], wn1b_ref[...],
                              preferred_element_type=jnp.float32)
    h = jnp.maximum(h + cst[0:1, :] + bn1_ref[...], 0.0)
    xn = x_ref[...] + jnp.dot(h, wn2_ref[...],
                              preferred_element_type=jnp.float32) + bn2_ref[...]
    xnew_ref[...] = xn

    @pl.when(i == 0)
    def _():
        nsum_acc[...] = jnp.zeros_like(nsum_acc)

    nsum_acc[...] += jnp.sum(xn, axis=0, keepdims=True)

    @pl.when(i == GN - 1)
    def _():
        nmean = jnp.broadcast_to(nsum_acc[...] * (1.0 / N), (8, D))
        emean = jnp.broadcast_to(esum_ref[...] * (1.0 / E), (8, D))
        g8 = jnp.concatenate([nmean, emean, u8_ref[...]], axis=1)
        hg = jnp.maximum(
            jnp.dot(g8, wg1_ref[...], preferred_element_type=jnp.float32)
            + bg1_ref[...], 0.0)
        un = jnp.dot(hg, wg2_ref[...],
                     preferred_element_type=jnp.float32) + bg2_ref[...]
        unew_ref[...] = u8_ref[0:1, :] + un[0:1, :]


def _node_call(u8, wn1b, wn1c, bn1, wn2, bn2, esum, wg1, bg1, wg2, bg2,
               x, agg, qa):
    blk = lambda i: (i, 0)
    fixed = lambda i: (0, 0)
    return pl.pallas_call(
        _node_body,
        grid=(GN,),
        in_specs=[
            pl.BlockSpec((8, D), fixed),
            pl.BlockSpec((D, D), fixed),
            pl.BlockSpec((D, D), fixed),
            pl.BlockSpec((1, D), fixed),
            pl.BlockSpec((D, D), fixed),
            pl.BlockSpec((1, D), fixed),
            pl.BlockSpec((1, D), fixed),
            pl.BlockSpec((3 * D, D), fixed),
            pl.BlockSpec((1, D), fixed),
            pl.BlockSpec((D, D), fixed),
            pl.BlockSpec((1, D), fixed),
            pl.BlockSpec((BN, D), blk),
            pl.BlockSpec((BN, D), blk),
            pl.BlockSpec((BN, D), blk),
        ],
        out_specs=[pl.BlockSpec((BN, D), blk), pl.BlockSpec((1, D), fixed)],
        out_shape=[jax.ShapeDtypeStruct((N, D), jnp.float32),
                   jax.ShapeDtypeStruct((1, D), jnp.float32)],
        scratch_shapes=[pltpu.VMEM((1, D), jnp.float32)],
    )(u8, wn1b, wn1c, bn1, wn2, bn2, esum, wg1, bg1, wg2, bg2, x, agg, qa)


def kernel(x, edge_attr, u, edge_index, batch,
           We1, be1, We2, be2,
           Wn1, bn1, Wn2, bn2,
           Wg1, bg1, Wg2, bg2):
    src = edge_index[0].astype(jnp.int32)
    dst = edge_index[1].astype(jnp.int32)
    src2 = src.reshape(NCHUNK, CH)
    dst2 = dst.reshape(NCHUNK, CH)

    wcat = jnp.concatenate([We1[:D], We1[D:2 * D], Wn1[:D]], axis=1)
    u8 = jnp.broadcast_to(u, (8, D))
    be1r = be1.reshape(1, D)
    be2r = be2.reshape(1, D)
    bn1r = bn1.reshape(1, D)
    bn2r = bn2.reshape(1, D)
    bg1r = bg1.reshape(1, D)
    bg2r = bg2.reshape(1, D)
    zeros = jnp.zeros((ROWB, DH), jnp.float32)

    pa, pb, qa = _proj_call(x, wcat)
    g1, g2 = _gather_call(pa, pb, src2, dst2)
    edge_new, esum = _edge_call(u8, We1[2 * D:3 * D], We1[3 * D:], be1r,
                                We2, be2r, g1, g2, edge_attr)
    agg = _scatter_call(edge_new, dst2, zeros)
    x_new, u_new = _node_call(u8, Wn1[D:2 * D], Wn1[2 * D:], bn1r, Wn2, bn2r,
                              esum, Wg1, bg1r, Wg2, bg2r, x, agg, qa)
    return x_new, edge_new, u_new


# speedup vs baseline: 5.5858x; 1.3173x over previous
import functools

import jax
import jax.numpy as jnp
from jax import lax
from jax.experimental import pallas as pl
from jax.experimental.pallas import tpu as pltpu
from jax.experimental.pallas import tpu_sc as plsc

N = 10000
E = 160000
D = 256

NC = 2
NS = 16
NW = NC * NS

CH = 128
NCHUNK = E // CH
NCHPAD = NW * 40
KG = NCHPAD // NW
KS = NCHPAD // NS
DH = D // NC
ROWB = 640
NPAD = NS * ROWB

_MESH = dict(core_axis_name="c", subcore_axis_name="s", num_cores=NC,
             num_subcores=NS)


def _gather_body(pa_hbm, pb_hbm, src_hbm, dst_hbm, g1_hbm, g2_hbm,
                 idxs_v, idxd_v, rows1_v, rows2_v,
                 gsem0, gsem1, wsem0, wsem1):
    wid = lax.axis_index("s") * NC + lax.axis_index("c")
    base = pl.multiple_of(wid * KG, 8)
    gsem = (gsem0, gsem1)
    wsem = (wsem0, wsem1)

    pltpu.sync_copy(src_hbm.at[pl.ds(base, KG)], idxs_v)
    pltpu.sync_copy(dst_hbm.at[pl.ds(base, KG)], idxd_v)

    def start(k, slot):
        pltpu.async_copy(pa_hbm.at[idxs_v.at[k]], rows1_v.at[slot], gsem[slot])
        pltpu.async_copy(pb_hbm.at[idxd_v.at[k]], rows2_v.at[slot], gsem[slot])

    def drain_gather(slot):
        pltpu.make_async_copy(pa_hbm.at[pl.ds(0, CH)], rows1_v.at[slot],
                              gsem[slot]).wait()
        pltpu.make_async_copy(pb_hbm.at[pl.ds(0, CH)], rows2_v.at[slot],
                              gsem[slot]).wait()

    def write(k, slot):
        off = pl.multiple_of((base + k) * CH, CH)
        pltpu.async_copy(rows1_v.at[slot], g1_hbm.at[pl.ds(off, CH)],
                         wsem[slot])
        pltpu.async_copy(rows2_v.at[slot], g2_hbm.at[pl.ds(off, CH)],
                         wsem[slot])

    def drain_write(slot):
        pltpu.make_async_copy(rows1_v.at[slot],
                              g1_hbm.at[pl.ds(0, CH)], wsem[slot]).wait()
        pltpu.make_async_copy(rows2_v.at[slot],
                              g2_hbm.at[pl.ds(0, CH)], wsem[slot]).wait()

    def real(k):
        return base + k < NCHUNK

    @pl.when(real(0))
    def _():
        start(0, 0)

    def step(j, carry):
        for slot in (0, 1):
            k = 2 * j + slot
            other = 1 - slot
            nxt = (k + 1 < KG) & real(k + 1)

            @pl.when((k >= 1) & nxt)
            def _():
                drain_write(other)

            @pl.when(nxt)
            def _():
                start(k + 1, other)

            @pl.when(real(k))
            def _():
                drain_gather(slot)
                write(k, slot)

        return carry

    lax.fori_loop(0, KG // 2, step, 0)
    @pl.when(real(0))
    def _():
        drain_write(0)

    @pl.when(real(1))
    def _():
        drain_write(1)


_gather_call = functools.partial(
    pl.kernel,
    out_type=[jax.ShapeDtypeStruct((E, D // 2), jnp.int32),
              jax.ShapeDtypeStruct((E, D // 2), jnp.int32)],
    mesh=plsc.VectorSubcoreMesh(**_MESH),
    scratch_types=[
        pltpu.VMEM((KG, CH), jnp.int32),
        pltpu.VMEM((KG, CH), jnp.int32),
        pltpu.VMEM((2, CH, D // 2), jnp.int32),
        pltpu.VMEM((2, CH, D // 2), jnp.int32),
        pltpu.SemaphoreType.DMA,
        pltpu.SemaphoreType.DMA,
        pltpu.SemaphoreType.DMA,
        pltpu.SemaphoreType.DMA,
    ],
)(_gather_body)


def _scatter_body(enew_hbm, dst_hbm, zeros_hbm, agg_hbm, acc_sh, idx_v, pay_v,
                  lsem0, lsem1):
    cid = lax.axis_index("c")
    sid = lax.axis_index("s")
    lsem = (lsem0, lsem1)
    coff = pl.multiple_of(cid * DH, DH)
    roff = pl.multiple_of(sid * ROWB, ROWB)
    base = pl.multiple_of(sid * KS, 8)

    def real(k):
        return base + k < NCHUNK

    def load(k, slot):
        off = pl.multiple_of((base + k) * CH, CH)
        pltpu.async_copy(enew_hbm.at[pl.ds(off, CH), pl.ds(coff, DH)],
                         pay_v.at[slot], lsem[slot])

    def drain_load(slot):
        pltpu.make_async_copy(enew_hbm.at[pl.ds(0, CH), pl.ds(0, DH)],
                              pay_v.at[slot], lsem[slot]).wait()

    pltpu.sync_copy(dst_hbm.at[pl.ds(base, KS)], idx_v)

    @pl.when(real(0))
    def _():
        load(0, 0)

    @pl.when(real(1))
    def _():
        load(1, 1)

    @pl.when(sid < NS - 1)
    def _():
        pltpu.sync_copy(zeros_hbm, acc_sh.at[pl.ds(roff, ROWB)])

    @pl.when(sid == NS - 1)
    def _():
        pltpu.sync_copy(zeros_hbm.at[pl.ds(0, N - (NS - 1) * ROWB)],
                        acc_sh.at[pl.ds((NS - 1) * ROWB,
                                        N - (NS - 1) * ROWB)])

    plsc.subcore_barrier()

    def sstep(j, carry):
        for slot in (0, 1):
            k = 2 * j + slot

            @pl.when(real(k))
            def _():
                drain_load(slot)
                pltpu.sync_copy(pay_v.at[slot], acc_sh.at[idx_v.at[k]],
                                add=True)

            @pl.when((k + 2 < KS) & real(k + 2))
            def _():
                load(k + 2, slot)

        return carry

    lax.fori_loop(0, KS // 2, sstep, 0)
    plsc.subcore_barrier()

    @pl.when(sid < NS - 1)
    def _():
        pltpu.sync_copy(acc_sh.at[pl.ds(roff, ROWB)],
                        agg_hbm.at[pl.ds(roff, ROWB), pl.ds(coff, DH)])

    @pl.when(sid == NS - 1)
    def _():
        pltpu.sync_copy(acc_sh.at[pl.ds((NS - 1) * ROWB, N - (NS - 1) * ROWB)],
                        agg_hbm.at[pl.ds((NS - 1) * ROWB, N - (NS - 1) * ROWB),
                                   pl.ds(coff, DH)])


_scatter_call = functools.partial(
    pl.kernel,
    out_type=jax.ShapeDtypeStruct((N, D), jnp.float32),
    mesh=plsc.VectorSubcoreMesh(**_MESH),
    scratch_types=[
        pltpu.VMEM_SHARED((NPAD, DH), jnp.float32),
        pltpu.VMEM((KS, CH), jnp.int32),
        pltpu.VMEM((2, CH, DH), jnp.float32),
        pltpu.SemaphoreType.DMA,
        pltpu.SemaphoreType.DMA,
    ],
)(_scatter_body)


BN = 1000
GN = N // BN


def _pack_bf16_pair(lo_f32, hi_f32):
    tl = lax.bitcast_convert_type(lo_f32, jnp.uint32) + jnp.uint32(0x8000)
    th = lax.bitcast_convert_type(hi_f32, jnp.uint32) + jnp.uint32(0x8000)
    packed = lax.shift_right_logical(tl, jnp.uint32(16)) | (th & jnp.uint32(0xFFFF0000))
    return lax.bitcast_convert_type(packed, jnp.int32)


def _unpack_bf16_pair(packed_i32):
    u = lax.bitcast_convert_type(packed_i32, jnp.uint32)
    lo = lax.bitcast_convert_type(lax.shift_left(u, jnp.uint32(16)), jnp.float32)
    hi = lax.bitcast_convert_type(u & jnp.uint32(0xFFFF0000), jnp.float32)
    return lo, hi


def _proj_body(x_ref, w_ref, pa_ref, pb_ref, qa_ref):
    p = jnp.dot(x_ref[...], w_ref[...], preferred_element_type=jnp.float32)
    pa_ref[...] = _pack_bf16_pair(p[:, :D // 2], p[:, D // 2:D])
    pb_ref[...] = _pack_bf16_pair(p[:, D:D + D // 2], p[:, D + D // 2:2 * D])
    qa_ref[...] = p[:, 2 * D:]


def _proj_call(x, wcat):
    blk = lambda i: (i, 0)
    outp = jax.ShapeDtypeStruct((N, D // 2), jnp.int32)
    return pl.pallas_call(
        _proj_body,
        grid=(GN,),
        in_specs=[
            pl.BlockSpec((BN, D), blk),
            pl.BlockSpec((D, 3 * D), lambda i: (0, 0)),
        ],
        out_specs=[pl.BlockSpec((BN, D // 2), blk),
                   pl.BlockSpec((BN, D // 2), blk),
                   pl.BlockSpec((BN, D), blk)],
        out_shape=[outp, outp, jax.ShapeDtypeStruct((N, D), jnp.float32)],
    )(x, wcat)


BE = 1000
GE = E // BE


def _edge_body(u8_ref, we1c_ref, we1d_ref, be1_ref, we2_ref, be2_ref,
               g1_ref, g2_ref, ea_ref, enew_ref, esum_ref):
    i = pl.program_id(0)
    ea = ea_ref[...]
    cst = jnp.dot(u8_ref[...], we1d_ref[...], preferred_element_type=jnp.float32)
    lo1, hi1 = _unpack_bf16_pair(g1_ref[...])
    lo2, hi2 = _unpack_bf16_pair(g2_ref[...])
    g = jnp.concatenate([lo1 + lo2, hi1 + hi2], axis=-1)
    h = g + jnp.dot(ea, we1c_ref[...], preferred_element_type=jnp.float32)
    h = jnp.maximum(h + cst[0:1, :] + be1_ref[...], 0.0)
    enew = ea + jnp.dot(h, we2_ref[...],
                        preferred_element_type=jnp.float32) + be2_ref[...]
    enew_ref[...] = enew

    @pl.when(i == 0)
    def _():
        esum_ref[...] = jnp.zeros_like(esum_ref)

    esum_ref[...] += jnp.sum(enew, axis=0, keepdims=True)


def _edge_call(u8, we1c, we1d, be1, we2, be2, g1, g2, ea):
    blk = lambda i: (i, 0)
    fixed = lambda i: (0, 0)
    return pl.pallas_call(
        _edge_body,
        grid=(GE,),
        in_specs=[
            pl.BlockSpec((8, D), fixed),
            pl.BlockSpec((D, D), fixed),
            pl.BlockSpec((D, D), fixed),
            pl.BlockSpec((1, D), fixed),
            pl.BlockSpec((D, D), fixed),
            pl.BlockSpec((1, D), fixed),
            pl.BlockSpec((BE, D // 2), blk),
            pl.BlockSpec((BE, D // 2), blk),
            pl.BlockSpec((BE, D), blk),
        ],
        out_specs=[pl.BlockSpec((BE, D), blk), pl.BlockSpec((1, D), fixed)],
        out_shape=[jax.ShapeDtypeStruct((E, D), jnp.float32),
                   jax.ShapeDtypeStruct((1, D), jnp.float32)],
    )(u8, we1c, we1d, be1, we2, be2, g1, g2, ea)


def _node_body(u8_ref, wn1b_ref, wn1c_ref, bn1_ref, wn2_ref, bn2_ref,
               esum_ref, wg1_ref, bg1_ref, wg2_ref, bg2_ref,
               x_ref, agg_ref, qa_ref, xnew_ref, unew_ref, nsum_acc):
    i = pl.program_id(0)
    cst = jnp.dot(u8_ref[...], wn1c_ref[...], preferred_element_type=jnp.float32)
    h = qa_ref[...] + jnp.dot(agg_ref[...], wn1b_ref[...],
                              preferred_element_type=jnp.float32)
    h = jnp.maximum(h + cst[0:1, :] + bn1_ref[...], 0.0)
    xn = x_ref[...] + jnp.dot(h, wn2_ref[...],
                              preferred_element_type=jnp.float32) + bn2_ref[...]
    xnew_ref[...] = xn

    @pl.when(i == 0)
    def _():
        nsum_acc[...] = jnp.zeros_like(nsum_acc)

    nsum_acc[...] += jnp.sum(xn, axis=0, keepdims=True)

    @pl.when(i == GN - 1)
    def _():
        nmean = jnp.broadcast_to(nsum_acc[...] * (1.0 / N), (8, D))
        emean = jnp.broadcast_to(esum_ref[...] * (1.0 / E), (8, D))
        g8 = jnp.concatenate([nmean, emean, u8_ref[...]], axis=1)
        hg = jnp.maximum(
            jnp.dot(g8, wg1_ref[...], preferred_element_type=jnp.float32)
            + bg1_ref[...], 0.0)
        un = jnp.dot(hg, wg2_ref[...],
                     preferred_element_type=jnp.float32) + bg2_ref[...]
        unew_ref[...] = u8_ref[0:1, :] + un[0:1, :]


def _node_call(u8, wn1b, wn1c, bn1, wn2, bn2, esum, wg1, bg1, wg2, bg2,
               x, agg, qa):
    blk = lambda i: (i, 0)
    fixed = lambda i: (0, 0)
    return pl.pallas_call(
        _node_body,
        grid=(GN,),
        in_specs=[
            pl.BlockSpec((8, D), fixed),
            pl.BlockSpec((D, D), fixed),
            pl.BlockSpec((D, D), fixed),
            pl.BlockSpec((1, D), fixed),
            pl.BlockSpec((D, D), fixed),
            pl.BlockSpec((1, D), fixed),
            pl.BlockSpec((1, D), fixed),
            pl.BlockSpec((3 * D, D), fixed),
            pl.BlockSpec((1, D), fixed),
            pl.BlockSpec((D, D), fixed),
            pl.BlockSpec((1, D), fixed),
            pl.BlockSpec((BN, D), blk),
            pl.BlockSpec((BN, D), blk),
            pl.BlockSpec((BN, D), blk),
        ],
        out_specs=[pl.BlockSpec((BN, D), blk), pl.BlockSpec((1, D), fixed)],
        out_shape=[jax.ShapeDtypeStruct((N, D), jnp.float32),
                   jax.ShapeDtypeStruct((1, D), jnp.float32)],
        scratch_shapes=[pltpu.VMEM((1, D), jnp.float32)],
    )(u8, wn1b, wn1c, bn1, wn2, bn2, esum, wg1, bg1, wg2, bg2, x, agg, qa)


def kernel(x, edge_attr, u, edge_index, batch,
           We1, be1, We2, be2,
           Wn1, bn1, Wn2, bn2,
           Wg1, bg1, Wg2, bg2):
    src = edge_index[0].astype(jnp.int32)
    dst = edge_index[1].astype(jnp.int32)
    pad = ((0, NCHPAD - NCHUNK), (0, 0))
    src2 = jnp.pad(src.reshape(NCHUNK, CH), pad)
    dst2 = jnp.pad(dst.reshape(NCHUNK, CH), pad)

    wcat = jnp.concatenate([We1[:D], We1[D:2 * D], Wn1[:D]], axis=1)
    u8 = jnp.broadcast_to(u, (8, D))
    be1r = be1.reshape(1, D)
    be2r = be2.reshape(1, D)
    bn1r = bn1.reshape(1, D)
    bn2r = bn2.reshape(1, D)
    bg1r = bg1.reshape(1, D)
    bg2r = bg2.reshape(1, D)
    zeros = jnp.zeros((ROWB, DH), jnp.float32)

    pa, pb, qa = _proj_call(x, wcat)
    g1, g2 = _gather_call(pa, pb, src2, dst2)
    edge_new, esum = _edge_call(u8, We1[2 * D:3 * D], We1[3 * D:], be1r,
                                We2, be2r, g1, g2, edge_attr)
    agg = _scatter_call(edge_new, dst2, zeros)
    x_new, u_new = _node_call(u8, Wn1[D:2 * D], Wn1[2 * D:], bn1r, Wn2, bn2r,
                              esum, Wg1, bg1r, Wg2, bg2r, x, agg, qa)
    return x_new, edge_new, u_new


# speedup vs baseline: 6.2001x; 1.1100x over previous
import functools

import jax
import jax.numpy as jnp
from jax import lax
from jax.experimental import pallas as pl
from jax.experimental.pallas import tpu as pltpu
from jax.experimental.pallas import tpu_sc as plsc

N = 10000
E = 160000
D = 256

NC = 2
NS = 16
NW = NC * NS

CH = 128
NCHUNK = E // CH
NCHPAD = NW * 40
KG = NCHPAD // NW
KS = NCHPAD // NS
DH = D // NC
ROWB = 640
NPAD = NS * ROWB

_MESH = dict(core_axis_name="c", subcore_axis_name="s", num_cores=NC,
             num_subcores=NS)


def _gather_body(pa_hbm, pb_hbm, src_hbm, dst_hbm, g1_hbm, g2_hbm,
                 idxs_v, idxd_v, rows1_v, rows2_v,
                 gsem0, gsem1, wsem0, wsem1):
    wid = lax.axis_index("s") * NC + lax.axis_index("c")
    base = pl.multiple_of(wid * KG, 8)
    gsem = (gsem0, gsem1)
    wsem = (wsem0, wsem1)

    pltpu.sync_copy(src_hbm.at[pl.ds(base, KG)], idxs_v)
    pltpu.sync_copy(dst_hbm.at[pl.ds(base, KG)], idxd_v)

    def start(k, slot):
        pltpu.async_copy(pa_hbm.at[idxs_v.at[k]], rows1_v.at[slot], gsem[slot])
        pltpu.async_copy(pb_hbm.at[idxd_v.at[k]], rows2_v.at[slot], gsem[slot])

    def drain_gather(slot):
        pltpu.make_async_copy(pa_hbm.at[pl.ds(0, CH)], rows1_v.at[slot],
                              gsem[slot]).wait()
        pltpu.make_async_copy(pb_hbm.at[pl.ds(0, CH)], rows2_v.at[slot],
                              gsem[slot]).wait()

    def write(k, slot):
        off = pl.multiple_of((base + k) * CH, CH)
        pltpu.async_copy(rows1_v.at[slot], g1_hbm.at[pl.ds(off, CH)],
                         wsem[slot])
        pltpu.async_copy(rows2_v.at[slot], g2_hbm.at[pl.ds(off, CH)],
                         wsem[slot])

    def drain_write(slot):
        pltpu.make_async_copy(rows1_v.at[slot],
                              g1_hbm.at[pl.ds(0, CH)], wsem[slot]).wait()
        pltpu.make_async_copy(rows2_v.at[slot],
                              g2_hbm.at[pl.ds(0, CH)], wsem[slot]).wait()

    def real(k):
        return base + k < NCHUNK

    @pl.when(real(0))
    def _():
        start(0, 0)

    def step(j, carry):
        for slot in (0, 1):
            k = 2 * j + slot
            other = 1 - slot
            nxt = (k + 1 < KG) & real(k + 1)

            @pl.when((k >= 1) & nxt)
            def _():
                drain_write(other)

            @pl.when(nxt)
            def _():
                start(k + 1, other)

            @pl.when(real(k))
            def _():
                drain_gather(slot)
                write(k, slot)

        return carry

    lax.fori_loop(0, KG // 2, step, 0)
    @pl.when(real(0))
    def _():
        drain_write(0)

    @pl.when(real(1))
    def _():
        drain_write(1)


_gather_call = functools.partial(
    pl.kernel,
    out_type=[jax.ShapeDtypeStruct((E, D // 2), jnp.int32),
              jax.ShapeDtypeStruct((E, D // 2), jnp.int32)],
    mesh=plsc.VectorSubcoreMesh(**_MESH),
    scratch_types=[
        pltpu.VMEM((KG, CH), jnp.int32),
        pltpu.VMEM((KG, CH), jnp.int32),
        pltpu.VMEM((2, CH, D // 2), jnp.int32),
        pltpu.VMEM((2, CH, D // 2), jnp.int32),
        pltpu.SemaphoreType.DMA,
        pltpu.SemaphoreType.DMA,
        pltpu.SemaphoreType.DMA,
        pltpu.SemaphoreType.DMA,
    ],
)(_gather_body)


NSLOT = 2


def _scatter_body(enew_hbm, dst_hbm, zeros_hbm, agg_hbm, acc_sh, idx_v, pay_v,
                  lsem0, lsem1):
    cid = lax.axis_index("c")
    sid = lax.axis_index("s")
    lsem = (lsem0, lsem1)
    coff = pl.multiple_of(cid * DH, DH)
    roff = pl.multiple_of(sid * ROWB, ROWB)
    base = pl.multiple_of(sid * KS, 8)

    def real(k):
        return base + k < NCHUNK

    def load(k, slot):
        off = pl.multiple_of((base + k) * CH, CH)
        pltpu.async_copy(enew_hbm.at[pl.ds(off, CH), pl.ds(coff, DH)],
                         pay_v.at[slot], lsem[slot])

    def drain_load(slot):
        pltpu.make_async_copy(enew_hbm.at[pl.ds(0, CH), pl.ds(0, DH)],
                              pay_v.at[slot], lsem[slot]).wait()

    pltpu.sync_copy(dst_hbm.at[pl.ds(base, KS)], idx_v)

    for s in range(NSLOT):
        @pl.when(real(s))
        def _():
            load(s, s)

    @pl.when(sid < NS - 1)
    def _():
        pltpu.sync_copy(zeros_hbm, acc_sh.at[pl.ds(roff, ROWB)])

    @pl.when(sid == NS - 1)
    def _():
        pltpu.sync_copy(zeros_hbm.at[pl.ds(0, N - (NS - 1) * ROWB)],
                        acc_sh.at[pl.ds((NS - 1) * ROWB,
                                        N - (NS - 1) * ROWB)])

    plsc.subcore_barrier()

    def sstep(j, carry):
        for slot in range(NSLOT):
            k = NSLOT * j + slot

            @pl.when(real(k))
            def _():
                drain_load(slot)
                pltpu.sync_copy(pay_v.at[slot], acc_sh.at[idx_v.at[k]],
                                add=True)

            @pl.when((k + NSLOT < KS) & real(k + NSLOT))
            def _():
                load(k + NSLOT, slot)

        return carry

    lax.fori_loop(0, KS // NSLOT, sstep, 0)
    plsc.subcore_barrier()

    @pl.when(sid < NS - 1)
    def _():
        pltpu.sync_copy(acc_sh.at[pl.ds(roff, ROWB)],
                        agg_hbm.at[pl.ds(roff, ROWB), pl.ds(coff, DH)])

    @pl.when(sid == NS - 1)
    def _():
        pltpu.sync_copy(acc_sh.at[pl.ds((NS - 1) * ROWB, N - (NS - 1) * ROWB)],
                        agg_hbm.at[pl.ds((NS - 1) * ROWB, N - (NS - 1) * ROWB),
                                   pl.ds(coff, DH)])


_scatter_call = functools.partial(
    pl.kernel,
    out_type=jax.ShapeDtypeStruct((N, D), jnp.float32),
    mesh=plsc.VectorSubcoreMesh(**_MESH),
    scratch_types=[
        pltpu.VMEM_SHARED((NPAD, DH), jnp.float32),
        pltpu.VMEM((KS, CH), jnp.int32),
        pltpu.VMEM((NSLOT, CH, DH), jnp.float32),
        pltpu.SemaphoreType.DMA,
        pltpu.SemaphoreType.DMA,
    ],
)(_scatter_body)


BN = 1000
GN = N // BN


def _pack_bf16_pair(lo_f32, hi_f32):
    tl = lax.bitcast_convert_type(lo_f32, jnp.uint32) + jnp.uint32(0x8000)
    th = lax.bitcast_convert_type(hi_f32, jnp.uint32) + jnp.uint32(0x8000)
    packed = lax.shift_right_logical(tl, jnp.uint32(16)) | (th & jnp.uint32(0xFFFF0000))
    return lax.bitcast_convert_type(packed, jnp.int32)


def _unpack_bf16_pair(packed_i32):
    u = lax.bitcast_convert_type(packed_i32, jnp.uint32)
    lo = lax.bitcast_convert_type(lax.shift_left(u, jnp.uint32(16)), jnp.float32)
    hi = lax.bitcast_convert_type(u & jnp.uint32(0xFFFF0000), jnp.float32)
    return lo, hi


def _proj_body(x_ref, w_ref, pa_ref, pb_ref, qa_ref):
    p = jnp.dot(x_ref[...], w_ref[...], preferred_element_type=jnp.float32)
    pa_ref[...] = _pack_bf16_pair(p[:, :D // 2], p[:, D // 2:D])
    pb_ref[...] = _pack_bf16_pair(p[:, D:D + D // 2], p[:, D + D // 2:2 * D])
    qa_ref[...] = p[:, 2 * D:]


def _proj_call(x, wcat):
    blk = lambda i: (i, 0)
    outp = jax.ShapeDtypeStruct((N, D // 2), jnp.int32)
    return pl.pallas_call(
        _proj_body,
        grid=(GN,),
        in_specs=[
            pl.BlockSpec((BN, D), blk),
            pl.BlockSpec((D, 3 * D), lambda i: (0, 0)),
        ],
        out_specs=[pl.BlockSpec((BN, D // 2), blk),
                   pl.BlockSpec((BN, D // 2), blk),
                   pl.BlockSpec((BN, D), blk)],
        out_shape=[outp, outp, jax.ShapeDtypeStruct((N, D), jnp.float32)],
    )(x, wcat)


BE = 2000
GE = E // BE


def _edge_body(u8_ref, we1c_ref, we1d_ref, be1_ref, we2_ref, be2_ref,
               g1_ref, g2_ref, ea_ref, enew_ref, esum_ref):
    i = pl.program_id(0)
    ea = ea_ref[...]
    cst = jnp.dot(u8_ref[...], we1d_ref[...], preferred_element_type=jnp.float32)
    lo1, hi1 = _unpack_bf16_pair(g1_ref[...])
    lo2, hi2 = _unpack_bf16_pair(g2_ref[...])
    g = jnp.concatenate([lo1 + lo2, hi1 + hi2], axis=-1)
    h = g + jnp.dot(ea, we1c_ref[...], preferred_element_type=jnp.float32)
    h = jnp.maximum(h + cst[0:1, :] + be1_ref[...], 0.0)
    enew = ea + jnp.dot(h, we2_ref[...],
                        preferred_element_type=jnp.float32) + be2_ref[...]
    enew_ref[...] = enew

    @pl.when(i == 0)
    def _():
        esum_ref[...] = jnp.zeros_like(esum_ref)

    esum_ref[...] += jnp.sum(enew, axis=0, keepdims=True)


def _edge_call(u8, we1c, we1d, be1, we2, be2, g1, g2, ea):
    blk = lambda i: (i, 0)
    fixed = lambda i: (0, 0)
    return pl.pallas_call(
        _edge_body,
        grid=(GE,),
        in_specs=[
            pl.BlockSpec((8, D), fixed),
            pl.BlockSpec((D, D), fixed),
            pl.BlockSpec((D, D), fixed),
            pl.BlockSpec((1, D), fixed),
            pl.BlockSpec((D, D), fixed),
            pl.BlockSpec((1, D), fixed),
            pl.BlockSpec((BE, D // 2), blk),
            pl.BlockSpec((BE, D // 2), blk),
            pl.BlockSpec((BE, D), blk),
        ],
        out_specs=[pl.BlockSpec((BE, D), blk), pl.BlockSpec((1, D), fixed)],
        out_shape=[jax.ShapeDtypeStruct((E, D), jnp.float32),
                   jax.ShapeDtypeStruct((1, D), jnp.float32)],
    )(u8, we1c, we1d, be1, we2, be2, g1, g2, ea)


def _node_body(u8_ref, wn1b_ref, wn1c_ref, bn1_ref, wn2_ref, bn2_ref,
               esum_ref, wg1_ref, bg1_ref, wg2_ref, bg2_ref,
               x_ref, agg_ref, qa_ref, xnew_ref, unew_ref, nsum_acc):
    i = pl.program_id(0)
    cst = jnp.dot(u8_ref[...], wn1c_ref[...], preferred_element_type=jnp.float32)
    h = qa_ref[...] + jnp.dot(agg_ref[...], wn1b_ref[...],
                              preferred_element_type=jnp.float32)
    h = jnp.maximum(h + cst[0:1, :] + bn1_ref[...], 0.0)
    xn = x_ref[...] + jnp.dot(h, wn2_ref[...],
                              preferred_element_type=jnp.float32) + bn2_ref[...]
    xnew_ref[...] = xn

    @pl.when(i == 0)
    def _():
        nsum_acc[...] = jnp.zeros_like(nsum_acc)

    nsum_acc[...] += jnp.sum(xn, axis=0, keepdims=True)

    @pl.when(i == GN - 1)
    def _():
        nmean = jnp.broadcast_to(nsum_acc[...] * (1.0 / N), (8, D))
        emean = jnp.broadcast_to(esum_ref[...] * (1.0 / E), (8, D))
        g8 = jnp.concatenate([nmean, emean, u8_ref[...]], axis=1)
        hg = jnp.maximum(
            jnp.dot(g8, wg1_ref[...], preferred_element_type=jnp.float32)
            + bg1_ref[...], 0.0)
        un = jnp.dot(hg, wg2_ref[...],
                     preferred_element_type=jnp.float32) + bg2_ref[...]
        unew_ref[...] = u8_ref[0:1, :] + un[0:1, :]


def _node_call(u8, wn1b, wn1c, bn1, wn2, bn2, esum, wg1, bg1, wg2, bg2,
               x, agg, qa):
    blk = lambda i: (i, 0)
    fixed = lambda i: (0, 0)
    return pl.pallas_call(
        _node_body,
        grid=(GN,),
        in_specs=[
            pl.BlockSpec((8, D), fixed),
            pl.BlockSpec((D, D), fixed),
            pl.BlockSpec((D, D), fixed),
            pl.BlockSpec((1, D), fixed),
            pl.BlockSpec((D, D), fixed),
            pl.BlockSpec((1, D), fixed),
            pl.BlockSpec((1, D), fixed),
            pl.BlockSpec((3 * D, D), fixed),
            pl.BlockSpec((1, D), fixed),
            pl.BlockSpec((D, D), fixed),
            pl.BlockSpec((1, D), fixed),
            pl.BlockSpec((BN, D), blk),
            pl.BlockSpec((BN, D), blk),
            pl.BlockSpec((BN, D), blk),
        ],
        out_specs=[pl.BlockSpec((BN, D), blk), pl.BlockSpec((1, D), fixed)],
        out_shape=[jax.ShapeDtypeStruct((N, D), jnp.float32),
                   jax.ShapeDtypeStruct((1, D), jnp.float32)],
        scratch_shapes=[pltpu.VMEM((1, D), jnp.float32)],
    )(u8, wn1b, wn1c, bn1, wn2, bn2, esum, wg1, bg1, wg2, bg2, x, agg, qa)


def kernel(x, edge_attr, u, edge_index, batch,
           We1, be1, We2, be2,
           Wn1, bn1, Wn2, bn2,
           Wg1, bg1, Wg2, bg2):
    src = edge_index[0].astype(jnp.int32)
    dst = edge_index[1].astype(jnp.int32)
    pad = ((0, NCHPAD - NCHUNK), (0, 0))
    src2 = jnp.pad(src.reshape(NCHUNK, CH), pad)
    dst2 = jnp.pad(dst.reshape(NCHUNK, CH), pad)

    wcat = jnp.concatenate([We1[:D], We1[D:2 * D], Wn1[:D]], axis=1)
    u8 = jnp.broadcast_to(u, (8, D))
    be1r = be1.reshape(1, D)
    be2r = be2.reshape(1, D)
    bn1r = bn1.reshape(1, D)
    bn2r = bn2.reshape(1, D)
    bg1r = bg1.reshape(1, D)
    bg2r = bg2.reshape(1, D)
    zeros = jnp.zeros((ROWB, DH), jnp.float32)

    pa, pb, qa = _proj_call(x, wcat)
    g1, g2 = _gather_call(pa, pb, src2, dst2)
    edge_new, esum = _edge_call(u8, We1[2 * D:3 * D], We1[3 * D:], be1r,
                                We2, be2r, g1, g2, edge_attr)
    agg = _scatter_call(edge_new, dst2, zeros)
    x_new, u_new = _node_call(u8, Wn1[D:2 * D], Wn1[2 * D:], bn1r, Wn2, bn2r,
                              esum, Wg1, bg1r, Wg2, bg2r, x, agg, qa)
    return x_new, edge_new, u_new


# speedup vs baseline: 6.5115x; 1.0502x over previous
import functools

import jax
import jax.numpy as jnp
from jax import lax
from jax.experimental import pallas as pl
from jax.experimental.pallas import tpu as pltpu
from jax.experimental.pallas import tpu_sc as plsc

N = 10000
E = 160000
D = 256

NC = 2
NS = 16
NW = NC * NS

CH = 128
NCHUNK = E // CH
NCHPAD = NW * 40
KG = NCHPAD // NW
KS = NCHPAD // NS
DH = D // NC
ROWB = 640
NPAD = NS * ROWB

_MESH = dict(core_axis_name="c", subcore_axis_name="s", num_cores=NC,
             num_subcores=NS)


def _gather_body(pa_hbm, pb_hbm, src_hbm, dst_hbm, g1_hbm, g2_hbm,
                 idxs_v, idxd_v, rows1_v, rows2_v,
                 gsem0, gsem1, wsem0, wsem1):
    wid = lax.axis_index("s") * NC + lax.axis_index("c")
    base = pl.multiple_of(wid * KG, 8)
    gsem = (gsem0, gsem1)
    wsem = (wsem0, wsem1)

    pltpu.sync_copy(src_hbm.at[pl.ds(base, KG)], idxs_v)
    pltpu.sync_copy(dst_hbm.at[pl.ds(base, KG)], idxd_v)

    def start(k, slot):
        pltpu.async_copy(pa_hbm.at[idxs_v.at[k]], rows1_v.at[slot], gsem[slot])
        pltpu.async_copy(pb_hbm.at[idxd_v.at[k]], rows2_v.at[slot], gsem[slot])

    def drain_gather(slot):
        pltpu.make_async_copy(pa_hbm.at[pl.ds(0, CH)], rows1_v.at[slot],
                              gsem[slot]).wait()
        pltpu.make_async_copy(pb_hbm.at[pl.ds(0, CH)], rows2_v.at[slot],
                              gsem[slot]).wait()

    def write(k, slot):
        off = pl.multiple_of((base + k) * CH, CH)
        pltpu.async_copy(rows1_v.at[slot], g1_hbm.at[pl.ds(off, CH)],
                         wsem[slot])
        pltpu.async_copy(rows2_v.at[slot], g2_hbm.at[pl.ds(off, CH)],
                         wsem[slot])

    def drain_write(slot):
        pltpu.make_async_copy(rows1_v.at[slot],
                              g1_hbm.at[pl.ds(0, CH)], wsem[slot]).wait()
        pltpu.make_async_copy(rows2_v.at[slot],
                              g2_hbm.at[pl.ds(0, CH)], wsem[slot]).wait()

    def real(k):
        return base + k < NCHUNK

    @pl.when(real(0))
    def _():
        start(0, 0)

    def step(j, carry):
        for slot in (0, 1):
            k = 2 * j + slot
            other = 1 - slot
            nxt = (k + 1 < KG) & real(k + 1)

            @pl.when((k >= 1) & nxt)
            def _():
                drain_write(other)

            @pl.when(nxt)
            def _():
                start(k + 1, other)

            @pl.when(real(k))
            def _():
                drain_gather(slot)
                write(k, slot)

        return carry

    lax.fori_loop(0, KG // 2, step, 0)
    @pl.when(real(0))
    def _():
        drain_write(0)

    @pl.when(real(1))
    def _():
        drain_write(1)


_gather_call = functools.partial(
    pl.kernel,
    out_type=[jax.ShapeDtypeStruct((E, D // 2), jnp.int32),
              jax.ShapeDtypeStruct((E, D // 2), jnp.int32)],
    mesh=plsc.VectorSubcoreMesh(**_MESH),
    scratch_types=[
        pltpu.VMEM((KG, CH), jnp.int32),
        pltpu.VMEM((KG, CH), jnp.int32),
        pltpu.VMEM((2, CH, D // 2), jnp.int32),
        pltpu.VMEM((2, CH, D // 2), jnp.int32),
        pltpu.SemaphoreType.DMA,
        pltpu.SemaphoreType.DMA,
        pltpu.SemaphoreType.DMA,
        pltpu.SemaphoreType.DMA,
    ],
)(_gather_body)


NSLOT = 2


def _scatter_body(enew_hbm, dst_hbm, zeros_hbm, agg_hbm, acc_sh, idx_v, pay_v,
                  lsem0, lsem1):
    cid = lax.axis_index("c")
    sid = lax.axis_index("s")
    lsem = (lsem0, lsem1)
    coff = pl.multiple_of(cid * DH, DH)
    roff = pl.multiple_of(sid * ROWB, ROWB)
    base = pl.multiple_of(sid * KS, 8)

    def real(k):
        return base + k < NCHUNK

    def load(k, slot):
        off = pl.multiple_of((base + k) * CH, CH)
        pltpu.async_copy(enew_hbm.at[pl.ds(off, CH), pl.ds(coff, DH)],
                         pay_v.at[slot], lsem[slot])

    def drain_load(slot):
        pltpu.make_async_copy(enew_hbm.at[pl.ds(0, CH), pl.ds(0, DH)],
                              pay_v.at[slot], lsem[slot]).wait()

    pltpu.sync_copy(dst_hbm.at[pl.ds(base, KS)], idx_v)

    for s in range(NSLOT):
        @pl.when(real(s))
        def _():
            load(s, s)

    @pl.when(sid < NS - 1)
    def _():
        pltpu.sync_copy(zeros_hbm, acc_sh.at[pl.ds(roff, ROWB)])

    @pl.when(sid == NS - 1)
    def _():
        pltpu.sync_copy(zeros_hbm.at[pl.ds(0, N - (NS - 1) * ROWB)],
                        acc_sh.at[pl.ds((NS - 1) * ROWB,
                                        N - (NS - 1) * ROWB)])

    plsc.subcore_barrier()

    def sstep(j, carry):
        for slot in range(NSLOT):
            k = NSLOT * j + slot

            @pl.when(real(k))
            def _():
                drain_load(slot)
                pltpu.sync_copy(pay_v.at[slot], acc_sh.at[idx_v.at[k]],
                                add=True)

            @pl.when((k + NSLOT < KS) & real(k + NSLOT))
            def _():
                load(k + NSLOT, slot)

        return carry

    lax.fori_loop(0, KS // NSLOT, sstep, 0)
    plsc.subcore_barrier()

    @pl.when(sid < NS - 1)
    def _():
        pltpu.sync_copy(acc_sh.at[pl.ds(roff, ROWB)],
                        agg_hbm.at[pl.ds(roff, ROWB), pl.ds(coff, DH)])

    @pl.when(sid == NS - 1)
    def _():
        pltpu.sync_copy(acc_sh.at[pl.ds((NS - 1) * ROWB, N - (NS - 1) * ROWB)],
                        agg_hbm.at[pl.ds((NS - 1) * ROWB, N - (NS - 1) * ROWB),
                                   pl.ds(coff, DH)])


_scatter_call = functools.partial(
    pl.kernel,
    out_type=jax.ShapeDtypeStruct((N, D), jnp.float32),
    mesh=plsc.VectorSubcoreMesh(**_MESH),
    scratch_types=[
        pltpu.VMEM_SHARED((NPAD, DH), jnp.float32),
        pltpu.VMEM((KS, CH), jnp.int32),
        pltpu.VMEM((NSLOT, CH, DH), jnp.float32),
        pltpu.SemaphoreType.DMA,
        pltpu.SemaphoreType.DMA,
    ],
)(_scatter_body)


BN = 2000
GN = N // BN


def _pack_bf16_pair(lo_f32, hi_f32):
    tl = lax.bitcast_convert_type(lo_f32, jnp.uint32) + jnp.uint32(0x8000)
    th = lax.bitcast_convert_type(hi_f32, jnp.uint32) + jnp.uint32(0x8000)
    packed = lax.shift_right_logical(tl, jnp.uint32(16)) | (th & jnp.uint32(0xFFFF0000))
    return lax.bitcast_convert_type(packed, jnp.int32)


def _unpack_bf16_pair(packed_i32):
    u = lax.bitcast_convert_type(packed_i32, jnp.uint32)
    lo = lax.bitcast_convert_type(lax.shift_left(u, jnp.uint32(16)), jnp.float32)
    hi = lax.bitcast_convert_type(u & jnp.uint32(0xFFFF0000), jnp.float32)
    return lo, hi


def _proj_body(x_ref, w_ref, pa_ref, pb_ref, qa_ref):
    p = jnp.dot(x_ref[...], w_ref[...], preferred_element_type=jnp.float32)
    pa_ref[...] = _pack_bf16_pair(p[:, :D // 2], p[:, D // 2:D])
    pb_ref[...] = _pack_bf16_pair(p[:, D:D + D // 2], p[:, D + D // 2:2 * D])
    qa_ref[...] = p[:, 2 * D:]


def _proj_call(x, wcat):
    blk = lambda i: (i, 0)
    outp = jax.ShapeDtypeStruct((N, D // 2), jnp.int32)
    return pl.pallas_call(
        _proj_body,
        grid=(GN,),
        in_specs=[
            pl.BlockSpec((BN, D), blk),
            pl.BlockSpec((D, 3 * D), lambda i: (0, 0)),
        ],
        out_specs=[pl.BlockSpec((BN, D // 2), blk),
                   pl.BlockSpec((BN, D // 2), blk),
                   pl.BlockSpec((BN, D), blk)],
        out_shape=[outp, outp, jax.ShapeDtypeStruct((N, D), jnp.float32)],
    )(x, wcat)


BE = 4000
GE = E // BE


def _edge_body(u8_ref, we1c_ref, we1d_ref, be1_ref, we2_ref, be2_ref,
               g1_ref, g2_ref, ea_ref, enew_ref, esum_ref):
    i = pl.program_id(0)
    ea = ea_ref[...]
    cst = jnp.dot(u8_ref[...], we1d_ref[...], preferred_element_type=jnp.float32)
    lo1, hi1 = _unpack_bf16_pair(g1_ref[...])
    lo2, hi2 = _unpack_bf16_pair(g2_ref[...])
    g = jnp.concatenate([lo1 + lo2, hi1 + hi2], axis=-1)
    h = g + jnp.dot(ea, we1c_ref[...], preferred_element_type=jnp.float32)
    h = jnp.maximum(h + cst[0:1, :] + be1_ref[...], 0.0)
    enew = ea + jnp.dot(h, we2_ref[...],
                        preferred_element_type=jnp.float32) + be2_ref[...]
    enew_ref[...] = enew

    @pl.when(i == 0)
    def _():
        esum_ref[...] = jnp.zeros_like(esum_ref)

    esum_ref[...] += jnp.sum(enew, axis=0, keepdims=True)


def _edge_call(u8, we1c, we1d, be1, we2, be2, g1, g2, ea):
    blk = lambda i: (i, 0)
    fixed = lambda i: (0, 0)
    return pl.pallas_call(
        _edge_body,
        grid=(GE,),
        in_specs=[
            pl.BlockSpec((8, D), fixed),
            pl.BlockSpec((D, D), fixed),
            pl.BlockSpec((D, D), fixed),
            pl.BlockSpec((1, D), fixed),
            pl.BlockSpec((D, D), fixed),
            pl.BlockSpec((1, D), fixed),
            pl.BlockSpec((BE, D // 2), blk),
            pl.BlockSpec((BE, D // 2), blk),
            pl.BlockSpec((BE, D), blk),
        ],
        out_specs=[pl.BlockSpec((BE, D), blk), pl.BlockSpec((1, D), fixed)],
        out_shape=[jax.ShapeDtypeStruct((E, D), jnp.float32),
                   jax.ShapeDtypeStruct((1, D), jnp.float32)],
    )(u8, we1c, we1d, be1, we2, be2, g1, g2, ea)


def _node_body(u8_ref, wn1b_ref, wn1c_ref, bn1_ref, wn2_ref, bn2_ref,
               esum_ref, wg1_ref, bg1_ref, wg2_ref, bg2_ref,
               x_ref, agg_ref, qa_ref, xnew_ref, unew_ref, nsum_acc):
    i = pl.program_id(0)
    cst = jnp.dot(u8_ref[...], wn1c_ref[...], preferred_element_type=jnp.float32)
    h = qa_ref[...] + jnp.dot(agg_ref[...], wn1b_ref[...],
                              preferred_element_type=jnp.float32)
    h = jnp.maximum(h + cst[0:1, :] + bn1_ref[...], 0.0)
    xn = x_ref[...] + jnp.dot(h, wn2_ref[...],
                              preferred_element_type=jnp.float32) + bn2_ref[...]
    xnew_ref[...] = xn

    @pl.when(i == 0)
    def _():
        nsum_acc[...] = jnp.zeros_like(nsum_acc)

    nsum_acc[...] += jnp.sum(xn, axis=0, keepdims=True)

    @pl.when(i == GN - 1)
    def _():
        nmean = jnp.broadcast_to(nsum_acc[...] * (1.0 / N), (8, D))
        emean = jnp.broadcast_to(esum_ref[...] * (1.0 / E), (8, D))
        g8 = jnp.concatenate([nmean, emean, u8_ref[...]], axis=1)
        hg = jnp.maximum(
            jnp.dot(g8, wg1_ref[...], preferred_element_type=jnp.float32)
            + bg1_ref[...], 0.0)
        un = jnp.dot(hg, wg2_ref[...],
                     preferred_element_type=jnp.float32) + bg2_ref[...]
        unew_ref[...] = u8_ref[0:1, :] + un[0:1, :]


def _node_call(u8, wn1b, wn1c, bn1, wn2, bn2, esum, wg1, bg1, wg2, bg2,
               x, agg, qa):
    blk = lambda i: (i, 0)
    fixed = lambda i: (0, 0)
    return pl.pallas_call(
        _node_body,
        grid=(GN,),
        in_specs=[
            pl.BlockSpec((8, D), fixed),
            pl.BlockSpec((D, D), fixed),
            pl.BlockSpec((D, D), fixed),
            pl.BlockSpec((1, D), fixed),
            pl.BlockSpec((D, D), fixed),
            pl.BlockSpec((1, D), fixed),
            pl.BlockSpec((1, D), fixed),
            pl.BlockSpec((3 * D, D), fixed),
            pl.BlockSpec((1, D), fixed),
            pl.BlockSpec((D, D), fixed),
            pl.BlockSpec((1, D), fixed),
            pl.BlockSpec((BN, D), blk),
            pl.BlockSpec((BN, D), blk),
            pl.BlockSpec((BN, D), blk),
        ],
        out_specs=[pl.BlockSpec((BN, D), blk), pl.BlockSpec((1, D), fixed)],
        out_shape=[jax.ShapeDtypeStruct((N, D), jnp.float32),
                   jax.ShapeDtypeStruct((1, D), jnp.float32)],
        scratch_shapes=[pltpu.VMEM((1, D), jnp.float32)],
    )(u8, wn1b, wn1c, bn1, wn2, bn2, esum, wg1, bg1, wg2, bg2, x, agg, qa)


def kernel(x, edge_attr, u, edge_index, batch,
           We1, be1, We2, be2,
           Wn1, bn1, Wn2, bn2,
           Wg1, bg1, Wg2, bg2):
    src = edge_index[0].astype(jnp.int32)
    dst = edge_index[1].astype(jnp.int32)
    pad = ((0, NCHPAD - NCHUNK), (0, 0))
    src2 = jnp.pad(src.reshape(NCHUNK, CH), pad)
    dst2 = jnp.pad(dst.reshape(NCHUNK, CH), pad)

    wcat = jnp.concatenate([We1[:D], We1[D:2 * D], Wn1[:D]], axis=1)
    u8 = jnp.broadcast_to(u, (8, D))
    be1r = be1.reshape(1, D)
    be2r = be2.reshape(1, D)
    bn1r = bn1.reshape(1, D)
    bn2r = bn2.reshape(1, D)
    bg1r = bg1.reshape(1, D)
    bg2r = bg2.reshape(1, D)
    zeros = jnp.zeros((ROWB, DH), jnp.float32)

    pa, pb, qa = _proj_call(x, wcat)
    g1, g2 = _gather_call(pa, pb, src2, dst2)
    edge_new, esum = _edge_call(u8, We1[2 * D:3 * D], We1[3 * D:], be1r,
                                We2, be2r, g1, g2, edge_attr)
    agg = _scatter_call(edge_new, dst2, zeros)
    x_new, u_new = _node_call(u8, Wn1[D:2 * D], Wn1[2 * D:], bn1r, Wn2, bn2r,
                              esum, Wg1, bg1r, Wg2, bg2r, x, agg, qa)
    return x_new, edge_new, u_new


# speedup vs baseline: 6.5435x; 1.0049x over previous
import functools

import jax
import jax.numpy as jnp
from jax import lax
from jax.experimental import pallas as pl
from jax.experimental.pallas import tpu as pltpu
from jax.experimental.pallas import tpu_sc as plsc

N = 10000
E = 160000
D = 256

NC = 2
NS = 16
NW = NC * NS

CH = 128
NCHUNK = E // CH
NCHPAD = NW * 40
KG = NCHPAD // NW
KS = NCHPAD // NS
DH = D // NC
ROWB = 640
NPAD = NS * ROWB

_MESH = dict(core_axis_name="c", subcore_axis_name="s", num_cores=NC,
             num_subcores=NS)


def _gather_body(pa_hbm, pb_hbm, src_hbm, dst_hbm, g1_hbm, g2_hbm,
                 idxs_v, idxd_v, rows1_v, rows2_v,
                 gsem0, gsem1, wsem0, wsem1):
    wid = lax.axis_index("s") * NC + lax.axis_index("c")
    base = pl.multiple_of(wid * KG, 8)
    gsem = (gsem0, gsem1)
    wsem = (wsem0, wsem1)

    pltpu.sync_copy(src_hbm.at[pl.ds(base, KG)], idxs_v)
    pltpu.sync_copy(dst_hbm.at[pl.ds(base, KG)], idxd_v)

    def start(k, slot):
        pltpu.async_copy(pa_hbm.at[idxs_v.at[k]], rows1_v.at[slot], gsem[slot])
        pltpu.async_copy(pb_hbm.at[idxd_v.at[k]], rows2_v.at[slot], gsem[slot])

    def drain_gather(slot):
        pltpu.make_async_copy(pa_hbm.at[pl.ds(0, CH)], rows1_v.at[slot],
                              gsem[slot]).wait()
        pltpu.make_async_copy(pb_hbm.at[pl.ds(0, CH)], rows2_v.at[slot],
                              gsem[slot]).wait()

    def write(k, slot):
        off = pl.multiple_of((base + k) * CH, CH)
        pltpu.async_copy(rows1_v.at[slot], g1_hbm.at[pl.ds(off, CH)],
                         wsem[slot])
        pltpu.async_copy(rows2_v.at[slot], g2_hbm.at[pl.ds(off, CH)],
                         wsem[slot])

    def drain_write(slot):
        pltpu.make_async_copy(rows1_v.at[slot],
                              g1_hbm.at[pl.ds(0, CH)], wsem[slot]).wait()
        pltpu.make_async_copy(rows2_v.at[slot],
                              g2_hbm.at[pl.ds(0, CH)], wsem[slot]).wait()

    def real(k):
        return base + k < NCHUNK

    @pl.when(real(0))
    def _():
        start(0, 0)

    def step(j, carry):
        for slot in (0, 1):
            k = 2 * j + slot
            other = 1 - slot
            nxt = (k + 1 < KG) & real(k + 1)

            @pl.when((k >= 1) & nxt)
            def _():
                drain_write(other)

            @pl.when(nxt)
            def _():
                start(k + 1, other)

            @pl.when(real(k))
            def _():
                drain_gather(slot)
                write(k, slot)

        return carry

    lax.fori_loop(0, KG // 2, step, 0)
    @pl.when(real(0))
    def _():
        drain_write(0)

    @pl.when(real(1))
    def _():
        drain_write(1)


_gather_call = functools.partial(
    pl.kernel,
    out_type=[jax.ShapeDtypeStruct((E, D // 2), jnp.int32),
              jax.ShapeDtypeStruct((E, D // 2), jnp.int32)],
    mesh=plsc.VectorSubcoreMesh(**_MESH),
    scratch_types=[
        pltpu.VMEM((KG, CH), jnp.int32),
        pltpu.VMEM((KG, CH), jnp.int32),
        pltpu.VMEM((2, CH, D // 2), jnp.int32),
        pltpu.VMEM((2, CH, D // 2), jnp.int32),
        pltpu.SemaphoreType.DMA,
        pltpu.SemaphoreType.DMA,
        pltpu.SemaphoreType.DMA,
        pltpu.SemaphoreType.DMA,
    ],
)(_gather_body)


NSLOT = 2


def _scatter_body(enew_hbm, dst_hbm, zeros_hbm, agg_hbm, acc_sh, idx_v, pay_v,
                  lsem0, lsem1):
    cid = lax.axis_index("c")
    sid = lax.axis_index("s")
    lsem = (lsem0, lsem1)
    coff = pl.multiple_of(cid * DH, DH)
    roff = pl.multiple_of(sid * ROWB, ROWB)
    base = pl.multiple_of(sid * KS, 8)

    def real(k):
        return base + k < NCHUNK

    def load(k, slot):
        off = pl.multiple_of((base + k) * CH, CH)
        pltpu.async_copy(enew_hbm.at[pl.ds(off, CH), pl.ds(coff, DH)],
                         pay_v.at[slot], lsem[slot])

    def drain_load(slot):
        pltpu.make_async_copy(enew_hbm.at[pl.ds(0, CH), pl.ds(0, DH)],
                              pay_v.at[slot], lsem[slot]).wait()

    pltpu.sync_copy(dst_hbm.at[pl.ds(base, KS)], idx_v)

    for s in range(NSLOT):
        @pl.when(real(s))
        def _():
            load(s, s)

    @pl.when(sid < NS - 1)
    def _():
        pltpu.sync_copy(zeros_hbm, acc_sh.at[pl.ds(roff, ROWB)])

    @pl.when(sid == NS - 1)
    def _():
        pltpu.sync_copy(zeros_hbm.at[pl.ds(0, N - (NS - 1) * ROWB)],
                        acc_sh.at[pl.ds((NS - 1) * ROWB,
                                        N - (NS - 1) * ROWB)])

    plsc.subcore_barrier()

    def sstep(j, carry):
        for slot in range(NSLOT):
            k = NSLOT * j + slot

            @pl.when(real(k))
            def _():
                drain_load(slot)
                pltpu.sync_copy(pay_v.at[slot], acc_sh.at[idx_v.at[k]],
                                add=True)

            @pl.when((k + NSLOT < KS) & real(k + NSLOT))
            def _():
                load(k + NSLOT, slot)

        return carry

    lax.fori_loop(0, KS // NSLOT, sstep, 0)
    plsc.subcore_barrier()

    @pl.when(sid < NS - 1)
    def _():
        pltpu.sync_copy(acc_sh.at[pl.ds(roff, ROWB)],
                        agg_hbm.at[pl.ds(roff, ROWB), pl.ds(coff, DH)])

    @pl.when(sid == NS - 1)
    def _():
        pltpu.sync_copy(acc_sh.at[pl.ds((NS - 1) * ROWB, N - (NS - 1) * ROWB)],
                        agg_hbm.at[pl.ds((NS - 1) * ROWB, N - (NS - 1) * ROWB),
                                   pl.ds(coff, DH)])


_scatter_call = functools.partial(
    pl.kernel,
    out_type=jax.ShapeDtypeStruct((N, D), jnp.float32),
    mesh=plsc.VectorSubcoreMesh(**_MESH),
    scratch_types=[
        pltpu.VMEM_SHARED((NPAD, DH), jnp.float32),
        pltpu.VMEM((KS, CH), jnp.int32),
        pltpu.VMEM((NSLOT, CH, DH), jnp.float32),
        pltpu.SemaphoreType.DMA,
        pltpu.SemaphoreType.DMA,
    ],
)(_scatter_body)


BN = 2000
GN = N // BN


def _pack_bf16_pair(lo_f32, hi_f32):
    tl = lax.bitcast_convert_type(lo_f32, jnp.uint32) + jnp.uint32(0x8000)
    th = lax.bitcast_convert_type(hi_f32, jnp.uint32) + jnp.uint32(0x8000)
    packed = lax.shift_right_logical(tl, jnp.uint32(16)) | (th & jnp.uint32(0xFFFF0000))
    return lax.bitcast_convert_type(packed, jnp.int32)


def _unpack_bf16_pair(packed_i32):
    u = lax.bitcast_convert_type(packed_i32, jnp.uint32)
    lo = lax.bitcast_convert_type(lax.shift_left(u, jnp.uint32(16)), jnp.float32)
    hi = lax.bitcast_convert_type(u & jnp.uint32(0xFFFF0000), jnp.float32)
    return lo, hi


def _proj_body(x_ref, w_ref, pa_ref, pb_ref, qa_ref):
    p = jnp.dot(x_ref[...], w_ref[...], preferred_element_type=jnp.float32)
    pa_ref[...] = _pack_bf16_pair(p[:, :D // 2], p[:, D // 2:D])
    pb_ref[...] = _pack_bf16_pair(p[:, D:D + D // 2], p[:, D + D // 2:2 * D])
    qa_ref[...] = p[:, 2 * D:]


def _proj_call(x, wcat):
    blk = lambda i: (i, 0)
    outp = jax.ShapeDtypeStruct((N, D // 2), jnp.int32)
    return pl.pallas_call(
        _proj_body,
        grid=(GN,),
        in_specs=[
            pl.BlockSpec((BN, D), blk),
            pl.BlockSpec((D, 3 * D), lambda i: (0, 0)),
        ],
        out_specs=[pl.BlockSpec((BN, D // 2), blk),
                   pl.BlockSpec((BN, D // 2), blk),
                   pl.BlockSpec((BN, D), blk)],
        out_shape=[outp, outp, jax.ShapeDtypeStruct((N, D), jnp.float32)],
    )(x, wcat)


BE = 8000
GE = E // BE


def _edge_body(u8_ref, we1c_ref, we1d_ref, be1_ref, we2_ref, be2_ref,
               g1_ref, g2_ref, ea_ref, enew_ref, esum_ref):
    i = pl.program_id(0)
    ea = ea_ref[...]
    cst = jnp.dot(u8_ref[...], we1d_ref[...], preferred_element_type=jnp.float32)
    lo1, hi1 = _unpack_bf16_pair(g1_ref[...])
    lo2, hi2 = _unpack_bf16_pair(g2_ref[...])
    g = jnp.concatenate([lo1 + lo2, hi1 + hi2], axis=-1)
    h = g + jnp.dot(ea, we1c_ref[...], preferred_element_type=jnp.float32)
    h = jnp.maximum(h + cst[0:1, :] + be1_ref[...], 0.0)
    enew = ea + jnp.dot(h, we2_ref[...],
                        preferred_element_type=jnp.float32) + be2_ref[...]
    enew_ref[...] = enew

    @pl.when(i == 0)
    def _():
        esum_ref[...] = jnp.zeros_like(esum_ref)

    esum_ref[...] += jnp.sum(enew, axis=0, keepdims=True)


def _edge_call(u8, we1c, we1d, be1, we2, be2, g1, g2, ea):
    blk = lambda i: (i, 0)
    fixed = lambda i: (0, 0)
    return pl.pallas_call(
        _edge_body,
        grid=(GE,),
        in_specs=[
            pl.BlockSpec((8, D), fixed),
            pl.BlockSpec((D, D), fixed),
            pl.BlockSpec((D, D), fixed),
            pl.BlockSpec((1, D), fixed),
            pl.BlockSpec((D, D), fixed),
            pl.BlockSpec((1, D), fixed),
            pl.BlockSpec((BE, D // 2), blk),
            pl.BlockSpec((BE, D // 2), blk),
            pl.BlockSpec((BE, D), blk),
        ],
        out_specs=[pl.BlockSpec((BE, D), blk), pl.BlockSpec((1, D), fixed)],
        out_shape=[jax.ShapeDtypeStruct((E, D), jnp.float32),
                   jax.ShapeDtypeStruct((1, D), jnp.float32)],
    )(u8, we1c, we1d, be1, we2, be2, g1, g2, ea)


def _node_body(u8_ref, wn1b_ref, wn1c_ref, bn1_ref, wn2_ref, bn2_ref,
               esum_ref, wg1_ref, bg1_ref, wg2_ref, bg2_ref,
               x_ref, agg_ref, qa_ref, xnew_ref, unew_ref, nsum_acc):
    i = pl.program_id(0)
    cst = jnp.dot(u8_ref[...], wn1c_ref[...], preferred_element_type=jnp.float32)
    h = qa_ref[...] + jnp.dot(agg_ref[...], wn1b_ref[...],
                              preferred_element_type=jnp.float32)
    h = jnp.maximum(h + cst[0:1, :] + bn1_ref[...], 0.0)
    xn = x_ref[...] + jnp.dot(h, wn2_ref[...],
                              preferred_element_type=jnp.float32) + bn2_ref[...]
    xnew_ref[...] = xn

    @pl.when(i == 0)
    def _():
        nsum_acc[...] = jnp.zeros_like(nsum_acc)

    nsum_acc[...] += jnp.sum(xn, axis=0, keepdims=True)

    @pl.when(i == GN - 1)
    def _():
        nmean = jnp.broadcast_to(nsum_acc[...] * (1.0 / N), (8, D))
        emean = jnp.broadcast_to(esum_ref[...] * (1.0 / E), (8, D))
        g8 = jnp.concatenate([nmean, emean, u8_ref[...]], axis=1)
        hg = jnp.maximum(
            jnp.dot(g8, wg1_ref[...], preferred_element_type=jnp.float32)
            + bg1_ref[...], 0.0)
        un = jnp.dot(hg, wg2_ref[...],
                     preferred_element_type=jnp.float32) + bg2_ref[...]
        unew_ref[...] = u8_ref[0:1, :] + un[0:1, :]


def _node_call(u8, wn1b, wn1c, bn1, wn2, bn2, esum, wg1, bg1, wg2, bg2,
               x, agg, qa):
    blk = lambda i: (i, 0)
    fixed = lambda i: (0, 0)
    return pl.pallas_call(
        _node_body,
        grid=(GN,),
        in_specs=[
            pl.BlockSpec((8, D), fixed),
            pl.BlockSpec((D, D), fixed),
            pl.BlockSpec((D, D), fixed),
            pl.BlockSpec((1, D), fixed),
            pl.BlockSpec((D, D), fixed),
            pl.BlockSpec((1, D), fixed),
            pl.BlockSpec((1, D), fixed),
            pl.BlockSpec((3 * D, D), fixed),
            pl.BlockSpec((1, D), fixed),
            pl.BlockSpec((D, D), fixed),
            pl.BlockSpec((1, D), fixed),
            pl.BlockSpec((BN, D), blk),
            pl.BlockSpec((BN, D), blk),
            pl.BlockSpec((BN, D), blk),
        ],
        out_specs=[pl.BlockSpec((BN, D), blk), pl.BlockSpec((1, D), fixed)],
        out_shape=[jax.ShapeDtypeStruct((N, D), jnp.float32),
                   jax.ShapeDtypeStruct((1, D), jnp.float32)],
        scratch_shapes=[pltpu.VMEM((1, D), jnp.float32)],
    )(u8, wn1b, wn1c, bn1, wn2, bn2, esum, wg1, bg1, wg2, bg2, x, agg, qa)


def kernel(x, edge_attr, u, edge_index, batch,
           We1, be1, We2, be2,
           Wn1, bn1, Wn2, bn2,
           Wg1, bg1, Wg2, bg2):
    src = edge_index[0].astype(jnp.int32)
    dst = edge_index[1].astype(jnp.int32)
    pad = ((0, NCHPAD - NCHUNK), (0, 0))
    src2 = jnp.pad(src.reshape(NCHUNK, CH), pad)
    dst2 = jnp.pad(dst.reshape(NCHUNK, CH), pad)

    wcat = jnp.concatenate([We1[:D], We1[D:2 * D], Wn1[:D]], axis=1)
    u8 = jnp.broadcast_to(u, (8, D))
    be1r = be1.reshape(1, D)
    be2r = be2.reshape(1, D)
    bn1r = bn1.reshape(1, D)
    bn2r = bn2.reshape(1, D)
    bg1r = bg1.reshape(1, D)
    bg2r = bg2.reshape(1, D)
    zeros = jnp.zeros((ROWB, DH), jnp.float32)

    pa, pb, qa = _proj_call(x, wcat)
    g1, g2 = _gather_call(pa, pb, src2, dst2)
    edge_new, esum = _edge_call(u8, We1[2 * D:3 * D], We1[3 * D:], be1r,
                                We2, be2r, g1, g2, edge_attr)
    agg = _scatter_call(edge_new, dst2, zeros)
    x_new, u_new = _node_call(u8, Wn1[D:2 * D], Wn1[2 * D:], bn1r, Wn2, bn2r,
                              esum, Wg1, bg1r, Wg2, bg2r, x, agg, qa)
    return x_new, edge_new, u_new


# speedup vs baseline: 6.5444x; 1.0001x over previous
import functools

import jax
import jax.numpy as jnp
from jax import lax
from jax.experimental import pallas as pl
from jax.experimental.pallas import tpu as pltpu
from jax.experimental.pallas import tpu_sc as plsc

N = 10000
E = 160000
D = 256

NC = 2
NS = 16
NW = NC * NS

CH = 128
NCHUNK = E // CH
NCHPAD = NW * 40
KG = NCHPAD // NW
KS = NCHPAD // NS
DH = D // NC
ROWB = 640
NPAD = NS * ROWB

_MESH = dict(core_axis_name="c", subcore_axis_name="s", num_cores=NC,
             num_subcores=NS)


def _gather_body(pa_hbm, pb_hbm, src_hbm, dst_hbm, g1_hbm, g2_hbm,
                 idxs_v, idxd_v, rows1_v, rows2_v,
                 gsem0, gsem1, gsem2, wsem0, wsem1, wsem2):
    wid = lax.axis_index("s") * NC + lax.axis_index("c")
    base = pl.multiple_of(wid * KG, 8)
    gsem = (gsem0, gsem1, gsem2)
    wsem = (wsem0, wsem1, wsem2)

    pltpu.sync_copy(src_hbm.at[pl.ds(base, KG)], idxs_v)
    pltpu.sync_copy(dst_hbm.at[pl.ds(base, KG)], idxd_v)

    def start(k, slot):
        pltpu.async_copy(pa_hbm.at[idxs_v.at[k]], rows1_v.at[slot], gsem[slot])
        pltpu.async_copy(pb_hbm.at[idxd_v.at[k]], rows2_v.at[slot], gsem[slot])

    def drain_gather(slot):
        pltpu.make_async_copy(pa_hbm.at[pl.ds(0, CH)], rows1_v.at[slot],
                              gsem[slot]).wait()
        pltpu.make_async_copy(pb_hbm.at[pl.ds(0, CH)], rows2_v.at[slot],
                              gsem[slot]).wait()

    def write(k, slot):
        off = pl.multiple_of((base + k) * CH, CH)
        pltpu.async_copy(rows1_v.at[slot], g1_hbm.at[pl.ds(off, CH)],
                         wsem[slot])
        pltpu.async_copy(rows2_v.at[slot], g2_hbm.at[pl.ds(off, CH)],
                         wsem[slot])

    def drain_write(slot):
        pltpu.make_async_copy(rows1_v.at[slot],
                              g1_hbm.at[pl.ds(0, CH)], wsem[slot]).wait()
        pltpu.make_async_copy(rows2_v.at[slot],
                              g2_hbm.at[pl.ds(0, CH)], wsem[slot]).wait()

    def real(k):
        return base + k < NCHUNK

    @pl.when(real(0))
    def _():
        start(0, 0)

    @pl.when(real(1))
    def _():
        start(1, 1)

    def substep(k, sidx):
        slot = sidx % 3
        nslot = (sidx + 2) % 3
        nxt2 = (k + 2 < KG) & real(k + 2)

        @pl.when((k >= 1) & nxt2)
        def _():
            drain_write(nslot)

        @pl.when(nxt2)
        def _():
            start(k + 2, nslot)

        @pl.when(real(k))
        def _():
            drain_gather(slot)
            write(k, slot)

    def step(j, carry):
        for s in range(3):
            substep(3 * j + s, s)
        return carry

    lax.fori_loop(0, KG // 3, step, 0)
    substep(KG - 1, (KG - 1) % 3)
    for s in range(3):
        @pl.when(real(s))
        def _():
            drain_write(s)


_gather_call = functools.partial(
    pl.kernel,
    out_type=[jax.ShapeDtypeStruct((E, D // 2), jnp.int32),
              jax.ShapeDtypeStruct((E, D // 2), jnp.int32)],
    mesh=plsc.VectorSubcoreMesh(**_MESH),
    scratch_types=[
        pltpu.VMEM((KG, CH), jnp.int32),
        pltpu.VMEM((KG, CH), jnp.int32),
        pltpu.VMEM((3, CH, D // 2), jnp.int32),
        pltpu.VMEM((3, CH, D // 2), jnp.int32),
        pltpu.SemaphoreType.DMA,
        pltpu.SemaphoreType.DMA,
        pltpu.SemaphoreType.DMA,
        pltpu.SemaphoreType.DMA,
        pltpu.SemaphoreType.DMA,
        pltpu.SemaphoreType.DMA,
    ],
)(_gather_body)


NSLOT = 2


def _scatter_body(enew_hbm, dst_hbm, zeros_hbm, agg_hbm, acc_sh, idx_v, pay_v,
                  lsem0, lsem1):
    cid = lax.axis_index("c")
    sid = lax.axis_index("s")
    lsem = (lsem0, lsem1)
    coff = pl.multiple_of(cid * DH, DH)
    roff = pl.multiple_of(sid * ROWB, ROWB)
    base = pl.multiple_of(sid * KS, 8)

    def real(k):
        return base + k < NCHUNK

    def load(k, slot):
        off = pl.multiple_of((base + k) * CH, CH)
        pltpu.async_copy(enew_hbm.at[pl.ds(off, CH), pl.ds(coff, DH)],
                         pay_v.at[slot], lsem[slot])

    def drain_load(slot):
        pltpu.make_async_copy(enew_hbm.at[pl.ds(0, CH), pl.ds(0, DH)],
                              pay_v.at[slot], lsem[slot]).wait()

    pltpu.sync_copy(dst_hbm.at[pl.ds(base, KS)], idx_v)

    for s in range(NSLOT):
        @pl.when(real(s))
        def _():
            load(s, s)

    @pl.when(sid < NS - 1)
    def _():
        pltpu.sync_copy(zeros_hbm, acc_sh.at[pl.ds(roff, ROWB)])

    @pl.when(sid == NS - 1)
    def _():
        pltpu.sync_copy(zeros_hbm.at[pl.ds(0, N - (NS - 1) * ROWB)],
                        acc_sh.at[pl.ds((NS - 1) * ROWB,
                                        N - (NS - 1) * ROWB)])

    plsc.subcore_barrier()

    def sstep(j, carry):
        for slot in range(NSLOT):
            k = NSLOT * j + slot

            @pl.when(real(k))
            def _():
                drain_load(slot)
                pltpu.sync_copy(pay_v.at[slot], acc_sh.at[idx_v.at[k]],
                                add=True)

            @pl.when((k + NSLOT < KS) & real(k + NSLOT))
            def _():
                load(k + NSLOT, slot)

        return carry

    lax.fori_loop(0, KS // NSLOT, sstep, 0)
    plsc.subcore_barrier()

    @pl.when(sid < NS - 1)
    def _():
        pltpu.sync_copy(acc_sh.at[pl.ds(roff, ROWB)],
                        agg_hbm.at[pl.ds(roff, ROWB), pl.ds(coff, DH)])

    @pl.when(sid == NS - 1)
    def _():
        pltpu.sync_copy(acc_sh.at[pl.ds((NS - 1) * ROWB, N - (NS - 1) * ROWB)],
                        agg_hbm.at[pl.ds((NS - 1) * ROWB, N - (NS - 1) * ROWB),
                                   pl.ds(coff, DH)])


_scatter_call = functools.partial(
    pl.kernel,
    out_type=jax.ShapeDtypeStruct((N, D), jnp.float32),
    mesh=plsc.VectorSubcoreMesh(**_MESH),
    scratch_types=[
        pltpu.VMEM_SHARED((NPAD, DH), jnp.float32),
        pltpu.VMEM((KS, CH), jnp.int32),
        pltpu.VMEM((NSLOT, CH, DH), jnp.float32),
        pltpu.SemaphoreType.DMA,
        pltpu.SemaphoreType.DMA,
    ],
)(_scatter_body)


BN = 2000
GN = N // BN


def _pack_bf16_pair(lo_f32, hi_f32):
    tl = lax.bitcast_convert_type(lo_f32, jnp.uint32) + jnp.uint32(0x8000)
    th = lax.bitcast_convert_type(hi_f32, jnp.uint32) + jnp.uint32(0x8000)
    packed = lax.shift_right_logical(tl, jnp.uint32(16)) | (th & jnp.uint32(0xFFFF0000))
    return lax.bitcast_convert_type(packed, jnp.int32)


def _unpack_bf16_pair(packed_i32):
    u = lax.bitcast_convert_type(packed_i32, jnp.uint32)
    lo = lax.bitcast_convert_type(lax.shift_left(u, jnp.uint32(16)), jnp.float32)
    hi = lax.bitcast_convert_type(u & jnp.uint32(0xFFFF0000), jnp.float32)
    return lo, hi


def _proj_body(x_ref, w_ref, pa_ref, pb_ref, qa_ref):
    p = jnp.dot(x_ref[...], w_ref[...], preferred_element_type=jnp.float32)
    pa_ref[...] = _pack_bf16_pair(p[:, :D // 2], p[:, D // 2:D])
    pb_ref[...] = _pack_bf16_pair(p[:, D:D + D // 2], p[:, D + D // 2:2 * D])
    qa_ref[...] = p[:, 2 * D:]


def _proj_call(x, wcat):
    blk = lambda i: (i, 0)
    outp = jax.ShapeDtypeStruct((N, D // 2), jnp.int32)
    return pl.pallas_call(
        _proj_body,
        grid=(GN,),
        in_specs=[
            pl.BlockSpec((BN, D), blk),
            pl.BlockSpec((D, 3 * D), lambda i: (0, 0)),
        ],
        out_specs=[pl.BlockSpec((BN, D // 2), blk),
                   pl.BlockSpec((BN, D // 2), blk),
                   pl.BlockSpec((BN, D), blk)],
        out_shape=[outp, outp, jax.ShapeDtypeStruct((N, D), jnp.float32)],
    )(x, wcat)


BE = 8000
GE = E // BE


def _edge_body(u8_ref, we1c_ref, we1d_ref, be1_ref, we2_ref, be2_ref,
               g1_ref, g2_ref, ea_ref, enew_ref, esum_ref):
    i = pl.program_id(0)
    ea = ea_ref[...]
    cst = jnp.dot(u8_ref[...], we1d_ref[...], preferred_element_type=jnp.float32)
    lo1, hi1 = _unpack_bf16_pair(g1_ref[...])
    lo2, hi2 = _unpack_bf16_pair(g2_ref[...])
    g = jnp.concatenate([lo1 + lo2, hi1 + hi2], axis=-1)
    h = g + jnp.dot(ea, we1c_ref[...], preferred_element_type=jnp.float32)
    h = jnp.maximum(h + cst[0:1, :] + be1_ref[...], 0.0)
    enew = ea + jnp.dot(h, we2_ref[...],
                        preferred_element_type=jnp.float32) + be2_ref[...]
    enew_ref[...] = enew

    @pl.when(i == 0)
    def _():
        esum_ref[...] = jnp.zeros_like(esum_ref)

    esum_ref[...] += jnp.sum(enew, axis=0, keepdims=True)


def _edge_call(u8, we1c, we1d, be1, we2, be2, g1, g2, ea):
    blk = lambda i: (i, 0)
    fixed = lambda i: (0, 0)
    return pl.pallas_call(
        _edge_body,
        grid=(GE,),
        in_specs=[
            pl.BlockSpec((8, D), fixed),
            pl.BlockSpec((D, D), fixed),
            pl.BlockSpec((D, D), fixed),
            pl.BlockSpec((1, D), fixed),
            pl.BlockSpec((D, D), fixed),
            pl.BlockSpec((1, D), fixed),
            pl.BlockSpec((BE, D // 2), blk),
            pl.BlockSpec((BE, D // 2), blk),
            pl.BlockSpec((BE, D), blk),
        ],
        out_specs=[pl.BlockSpec((BE, D), blk), pl.BlockSpec((1, D), fixed)],
        out_shape=[jax.ShapeDtypeStruct((E, D), jnp.float32),
                   jax.ShapeDtypeStruct((1, D), jnp.float32)],
    )(u8, we1c, we1d, be1, we2, be2, g1, g2, ea)


def _node_body(u8_ref, wn1b_ref, wn1c_ref, bn1_ref, wn2_ref, bn2_ref,
               esum_ref, wg1_ref, bg1_ref, wg2_ref, bg2_ref,
               x_ref, agg_ref, qa_ref, xnew_ref, unew_ref, nsum_acc):
    i = pl.program_id(0)
    cst = jnp.dot(u8_ref[...], wn1c_ref[...], preferred_element_type=jnp.float32)
    h = qa_ref[...] + jnp.dot(agg_ref[...], wn1b_ref[...],
                              preferred_element_type=jnp.float32)
    h = jnp.maximum(h + cst[0:1, :] + bn1_ref[...], 0.0)
    xn = x_ref[...] + jnp.dot(h, wn2_ref[...],
                              preferred_element_type=jnp.float32) + bn2_ref[...]
    xnew_ref[...] = xn

    @pl.when(i == 0)
    def _():
        nsum_acc[...] = jnp.zeros_like(nsum_acc)

    nsum_acc[...] += jnp.sum(xn, axis=0, keepdims=True)

    @pl.when(i == GN - 1)
    def _():
        nmean = jnp.broadcast_to(nsum_acc[...] * (1.0 / N), (8, D))
        emean = jnp.broadcast_to(esum_ref[...] * (1.0 / E), (8, D))
        g8 = jnp.concatenate([nmean, emean, u8_ref[...]], axis=1)
        hg = jnp.maximum(
            jnp.dot(g8, wg1_ref[...], preferred_element_type=jnp.float32)
            + bg1_ref[...], 0.0)
        un = jnp.dot(hg, wg2_ref[...],
                     preferred_element_type=jnp.float32) + bg2_ref[...]
        unew_ref[...] = u8_ref[0:1, :] + un[0:1, :]


def _node_call(u8, wn1b, wn1c, bn1, wn2, bn2, esum, wg1, bg1, wg2, bg2,
               x, agg, qa):
    blk = lambda i: (i, 0)
    fixed = lambda i: (0, 0)
    return pl.pallas_call(
        _node_body,
        grid=(GN,),
        in_specs=[
            pl.BlockSpec((8, D), fixed),
            pl.BlockSpec((D, D), fixed),
            pl.BlockSpec((D, D), fixed),
            pl.BlockSpec((1, D), fixed),
            pl.BlockSpec((D, D), fixed),
            pl.BlockSpec((1, D), fixed),
            pl.BlockSpec((1, D), fixed),
            pl.BlockSpec((3 * D, D), fixed),
            pl.BlockSpec((1, D), fixed),
            pl.BlockSpec((D, D), fixed),
            pl.BlockSpec((1, D), fixed),
            pl.BlockSpec((BN, D), blk),
            pl.BlockSpec((BN, D), blk),
            pl.BlockSpec((BN, D), blk),
        ],
        out_specs=[pl.BlockSpec((BN, D), blk), pl.BlockSpec((1, D), fixed)],
        out_shape=[jax.ShapeDtypeStruct((N, D), jnp.float32),
                   jax.ShapeDtypeStruct((1, D), jnp.float32)],
        scratch_shapes=[pltpu.VMEM((1, D), jnp.float32)],
    )(u8, wn1b, wn1c, bn1, wn2, bn2, esum, wg1, bg1, wg2, bg2, x, agg, qa)


def kernel(x, edge_attr, u, edge_index, batch,
           We1, be1, We2, be2,
           Wn1, bn1, Wn2, bn2,
           Wg1, bg1, Wg2, bg2):
    src = edge_index[0].astype(jnp.int32)
    dst = edge_index[1].astype(jnp.int32)
    pad = ((0, NCHPAD - NCHUNK), (0, 0))
    src2 = jnp.pad(src.reshape(NCHUNK, CH), pad)
    dst2 = jnp.pad(dst.reshape(NCHUNK, CH), pad)

    wcat = jnp.concatenate([We1[:D], We1[D:2 * D], Wn1[:D]], axis=1)
    u8 = jnp.broadcast_to(u, (8, D))
    be1r = be1.reshape(1, D)
    be2r = be2.reshape(1, D)
    bn1r = bn1.reshape(1, D)
    bn2r = bn2.reshape(1, D)
    bg1r = bg1.reshape(1, D)
    bg2r = bg2.reshape(1, D)
    zeros = jnp.zeros((ROWB, DH), jnp.float32)

    pa, pb, qa = _proj_call(x, wcat)
    g1, g2 = _gather_call(pa, pb, src2, dst2)
    edge_new, esum = _edge_call(u8, We1[2 * D:3 * D], We1[3 * D:], be1r,
                                We2, be2r, g1, g2, edge_attr)
    agg = _scatter_call(edge_new, dst2, zeros)
    x_new, u_new = _node_call(u8, Wn1[D:2 * D], Wn1[2 * D:], bn1r, Wn2, bn2r,
                              esum, Wg1, bg1r, Wg2, bg2r, x, agg, qa)
    return x_new, edge_new, u_new
